# baseline (speedup 1.0000x reference)
import functools

import jax
import jax.numpy as jnp
from jax import lax
from jax.experimental import pallas as pl
from jax.experimental.pallas import tpu as pltpu
from jax.experimental.pallas import tpu_sc as plsc

_C = 1000
_D = 256
_M = 4
_MOM = 0.9
_B = 16384
_CP = 1024
_NSEGP = _M * _CP
_H = _D // 128

_NC = 2
_NS = 16
_L = 16
_NW = _NC * _NS
_RPT = _B // _NW
_CHUNK = 128
_NCHUNK = _RPT // _CHUNK
_SEG_PT = _NSEGP // _NS


def _sc_body(feats_hbm, labels_hbm, domains_hbm, out_sums, out_cnts,
             fbuf, idx2d, lab_v, dom_v, seg_v, cnt_v, acc_sh):
    cid = lax.axis_index("c")
    sid = lax.axis_index("s")
    wid = sid * _NC + cid
    base = wid * _RPT

    zrow = jnp.zeros((_L,), jnp.float32)

    def _fill_row(i, carry):
        for k in range(128 // _L):
            fbuf[i, pl.ds(k * _L, _L)] = zrow
        cnt_v[pl.ds(i * _L, _L)] = zrow
        return carry

    lax.fori_loop(0, _H * _CHUNK, _fill_row, 0)

    r0 = sid * _SEG_PT * _H
    pltpu.sync_copy(fbuf, acc_sh.at[pl.ds(r0, _H * _CHUNK)])
    pltpu.sync_copy(fbuf, acc_sh.at[pl.ds(r0 + _H * _CHUNK, _H * _CHUNK)])

    pltpu.sync_copy(labels_hbm.at[pl.ds(base, _RPT)], lab_v)
    pltpu.sync_copy(domains_hbm.at[pl.ds(base, _RPT)], dom_v)
    onesv = zrow + 1.0
    for i in range(_RPT // _L):
        seg = dom_v[pl.ds(i * _L, _L)] * _CP + lab_v[pl.ds(i * _L, _L)]
        seg_v[pl.ds(i * _L, _L)] = seg
        plsc.addupdate_scatter(cnt_v, [seg], onesv)

    iota = lax.broadcasted_iota(jnp.int32, (_L,), 0)
    half = iota & 1
    rofs = iota >> 1
    for k in range(_H * _NCHUNK):
        for g8 in range(8):
            gg = k * 8 + g8
            segs = plsc.load_gather(seg_v, [gg * 8 + rofs])
            idx2d[k, pl.ds(g8 * _L, _L)] = segs * _H + half

    plsc.subcore_barrier()

    for j in range(_NCHUNK):
        pltpu.sync_copy(feats_hbm.at[pl.ds(_H * (base + j * _CHUNK), _H * _CHUNK)], fbuf)
        pltpu.sync_copy(fbuf.at[pl.ds(0, 128)], acc_sh.at[idx2d.at[2 * j]], add=True)
        pltpu.sync_copy(fbuf.at[pl.ds(128, 128)], acc_sh.at[idx2d.at[2 * j + 1]], add=True)

    plsc.subcore_barrier()

    pltpu.sync_copy(acc_sh.at[pl.ds(r0, _SEG_PT * _H)],
                    out_sums.at[cid, pl.ds(r0, _SEG_PT * _H)])
    pltpu.sync_copy(cnt_v, out_cnts.at[wid])


@functools.cache
def _get_sc_segsum():
    return pl.kernel(
        _sc_body,
        out_type=(
            jax.ShapeDtypeStruct((_NC, _NSEGP * _H, 128), jnp.float32),
            jax.ShapeDtypeStruct((_NW, _NSEGP), jnp.float32),
        ),
        mesh=plsc.VectorSubcoreMesh(core_axis_name="c", subcore_axis_name="s"),
        compiler_params=pltpu.CompilerParams(use_tc_tiling_on_sc=False,
                                             needs_layout_passes=False),
        scratch_types=[
            pltpu.VMEM((_H * _CHUNK, 128), jnp.float32),
            pltpu.VMEM((_H * _NCHUNK, 128), jnp.int32),
            pltpu.VMEM((_RPT,), jnp.int32),
            pltpu.VMEM((_RPT,), jnp.int32),
            pltpu.VMEM((_RPT,), jnp.int32),
            pltpu.VMEM((_NSEGP,), jnp.float32),
            pltpu.VMEM_SHARED((_NSEGP * _H, 128), jnp.float32),
        ],
    )


_BLK = 512


def _mm_body(dom_ref, x_ref, s2_ref):
    i = pl.program_id(0)

    @pl.when(i == 0)
    def _init():
        s2_ref[...] = jnp.zeros_like(s2_ref)

    x = x_ref[...].astype(jnp.bfloat16)
    dom = dom_ref[...]
    for d in range(_M):
        zd = jnp.where(dom == d, x, jnp.bfloat16(0.0))
        s2_ref[d] += lax.dot_general(
            zd, x, (((0,), (0,)), ((), ())), preferred_element_type=jnp.float32)


def _tc_moments(domain2d, feats):
    return pl.pallas_call(
        _mm_body,
        grid=(_B // _BLK,),
        in_specs=[
            pl.BlockSpec((_BLK, 1), lambda i: (i, 0)),
            pl.BlockSpec((_BLK, _D), lambda i: (i, 0)),
        ],
        out_specs=pl.BlockSpec((_M, _D, _D), lambda i: (0, 0, 0)),
        out_shape=jax.ShapeDtypeStruct((_M, _D, _D), jnp.float32),
        compiler_params=pltpu.CompilerParams(dimension_semantics=("arbitrary",)),
    )(domain2d, feats)


def _outer4(v):
    return v[:, :, None, None] * v[None, None, :, :]


def _ep_body(sums_ref, cnts_ref, s2_ref, out_ref):
    sums4 = (sums_ref[0] + sums_ref[1]).reshape(_M, _CP, _H, 128)
    cnts = (cnts_ref[...].reshape(_NW, _NSEGP // 128, 128).sum(axis=0)
            .reshape(_M, _CP // 128, 128).reshape(_M, _CP))

    csafe = jnp.maximum(cnts, 1.0)
    mu = sums4 / csafe[:, :, None, None]
    presf = (cnts > 0.0).astype(jnp.float32)

    anchors = (1.0 - _MOM) * mu * presf[:, :, None, None]
    ag = jnp.zeros((_CP, _H, 128), jnp.float32)
    for d in range(_M):
        upd = _MOM * ag + (1.0 - _MOM) * mu[d]
        pd = presf[d][:, None, None]
        ag = pd * upd + (1.0 - pd) * ag
    sq = (anchors - ag[None]) ** 2
    per = jnp.sum(jnp.sum(sq, axis=-1), axis=-1) / _D
    nvalid = jnp.sum(presf)
    caa = jnp.where(nvalid > 0,
                    jnp.sum(per * presf) / jnp.maximum(nvalid, 1.0),
                    0.0)

    s24 = s2_ref[...].reshape(_M, _H, 128, _H, 128)
    tot = jnp.sum(jnp.sum(sums4, axis=0), axis=0)
    mu_g = tot / _B
    s2_tot = jnp.sum(s24, axis=0)
    cov = (s2_tot - _B * _outer4(mu_g)) / (_B + 1e-6)
    i0 = lax.broadcasted_iota(jnp.int32, (_H, 128, _H, 128), 0)
    i1 = lax.broadcasted_iota(jnp.int32, (_H, 128, _H, 128), 1)
    i2 = lax.broadcasted_iota(jnp.int32, (_H, 128, _H, 128), 2)
    i3 = lax.broadcasted_iota(jnp.int32, (_H, 128, _H, 128), 3)
    eye = ((i0 == i2) & (i1 == i3)).astype(jnp.float32)
    g_mean = (1.0 - _MOM) * mu_g
    g_cov = _MOM * eye + (1.0 - _MOM) * cov

    loss = jnp.float32(0.0)
    nval = jnp.float32(0.0)
    for d in range(_M):
        cnt = jnp.sum(cnts[d])
        s_row = jnp.sum(sums4[d], axis=0)
        mu_d = s_row / jnp.maximum(cnt, 1.0)
        cov_d = (s24[d] - cnt * _outer4(mu_d)) / (cnt + 1e-6)
        l_d = jnp.mean((mu_d - g_mean) ** 2) + jnp.mean((cov_d - g_cov) ** 2)
        has = (cnt > 0).astype(jnp.float32)
        loss = loss + has * l_d
        nval = nval + has
    stats = jnp.where(nval > 0, loss / jnp.maximum(nval, 1.0), 0.0)

    out_ref[...] = jnp.full((1, 1), caa + stats, jnp.float32)


def _tc_epilogue(sums_p, cnts_p, s2):
    return pl.pallas_call(
        _ep_body,
        out_shape=jax.ShapeDtypeStruct((1, 1), jnp.float32),
    )(sums_p, cnts_p, s2)


def kernel(feats, labels, domain_ids):
    s2 = _tc_moments(domain_ids.reshape(_B, 1), feats)
    sums_p, cnts_p = _get_sc_segsum()(feats.reshape(_B * _H, 128),
                                      labels, domain_ids)
    loss = _tc_epilogue(sums_p,
                        cnts_p.reshape(_NW * _NSEGP // 128, 128),
                        s2)
    return loss.reshape(())


# speedup vs baseline: 1.3254x; 1.3254x over previous
import functools

import jax
import jax.numpy as jnp
from jax import lax
from jax.experimental import pallas as pl
from jax.experimental.pallas import tpu as pltpu
from jax.experimental.pallas import tpu_sc as plsc

_C = 1000
_D = 256
_M = 4
_MOM = 0.9
_B = 16384
_CP = 1024
_NSEGP = _M * _CP
_H = _D // 128

_NC = 2
_NS = 16
_L = 16
_NW = _NC * _NS
_RPT = _B // _NW
_CHUNK = 128
_NCHUNK = _RPT // _CHUNK
_SEG_PT = _NSEGP // _NS


def _sc_body(feats_hbm, labels_hbm, domains_hbm, out_sums, out_cnts,
             fbuf, idx2d, lab_v, dom_v, seg_v, cnt_v, acc_sh):
    cid = lax.axis_index("c")
    sid = lax.axis_index("s")
    wid = sid * _NC + cid
    base = wid * _RPT

    zrow = jnp.zeros((_L,), jnp.float32)

    def _fill_row(i, carry):
        for k in range(128 // _L):
            fbuf[i, pl.ds(k * _L, _L)] = zrow
        cnt_v[pl.ds(i * _L, _L)] = zrow
        return carry

    lax.fori_loop(0, _H * _CHUNK, _fill_row, 0)

    r0 = sid * _SEG_PT * _H
    pltpu.sync_copy(fbuf, acc_sh.at[pl.ds(r0, _H * _CHUNK)])
    pltpu.sync_copy(fbuf, acc_sh.at[pl.ds(r0 + _H * _CHUNK, _H * _CHUNK)])

    pltpu.sync_copy(labels_hbm.at[pl.ds(base, _RPT)], lab_v)
    pltpu.sync_copy(domains_hbm.at[pl.ds(base, _RPT)], dom_v)
    onesv = zrow + 1.0
    for i in range(_RPT // _L):
        seg = dom_v[pl.ds(i * _L, _L)] * _CP + lab_v[pl.ds(i * _L, _L)]
        seg_v[pl.ds(i * _L, _L)] = seg
        plsc.addupdate_scatter(cnt_v, [seg], onesv)

    iota = lax.broadcasted_iota(jnp.int32, (_L,), 0)
    half = iota & 1
    rofs = iota >> 1
    for k in range(_H * _NCHUNK):
        for g8 in range(8):
            gg = k * 8 + g8
            segs = plsc.load_gather(seg_v, [gg * 8 + rofs])
            idx2d[k, pl.ds(g8 * _L, _L)] = segs * _H + half

    plsc.subcore_barrier()

    for j in range(_NCHUNK):
        pltpu.sync_copy(feats_hbm.at[pl.ds(_H * (base + j * _CHUNK), _H * _CHUNK)], fbuf)
        pltpu.sync_copy(fbuf.at[pl.ds(0, 128)], acc_sh.at[idx2d.at[2 * j]], add=True)
        pltpu.sync_copy(fbuf.at[pl.ds(128, 128)], acc_sh.at[idx2d.at[2 * j + 1]], add=True)

    plsc.subcore_barrier()

    pltpu.sync_copy(acc_sh.at[pl.ds(r0, _SEG_PT * _H)],
                    out_sums.at[cid, pl.ds(r0, _SEG_PT * _H)])
    pltpu.sync_copy(cnt_v, out_cnts.at[wid])


@functools.cache
def _get_sc_segsum():
    return pl.kernel(
        _sc_body,
        out_type=(
            jax.ShapeDtypeStruct((_NC, _NSEGP * _H, 128), jnp.float32),
            jax.ShapeDtypeStruct((_NW, _NSEGP), jnp.float32),
        ),
        mesh=plsc.VectorSubcoreMesh(core_axis_name="c", subcore_axis_name="s"),
        compiler_params=pltpu.CompilerParams(use_tc_tiling_on_sc=False,
                                             needs_layout_passes=False),
        scratch_types=[
            pltpu.VMEM((_H * _CHUNK, 128), jnp.float32),
            pltpu.VMEM((_H * _NCHUNK, 128), jnp.int32),
            pltpu.VMEM((_RPT,), jnp.int32),
            pltpu.VMEM((_RPT,), jnp.int32),
            pltpu.VMEM((_RPT,), jnp.int32),
            pltpu.VMEM((_NSEGP,), jnp.float32),
            pltpu.VMEM_SHARED((_NSEGP * _H, 128), jnp.float32),
        ],
    )


_BLK = 1024


def _mm_body(dom_ref, x_ref, t_ref):
    i = pl.program_id(0)

    @pl.when(i == 0)
    def _init():
        t_ref[...] = jnp.zeros_like(t_ref)

    x = x_ref[...].astype(jnp.bfloat16)
    dom = dom_ref[...]
    xs1 = jnp.where((dom & 1) == 1, x, -x)
    xs2 = jnp.where((dom & 2) == 2, x, -x)
    dn = (((0,), (0,)), ((), ()))
    t_ref[0] += lax.dot_general(x, x, dn, preferred_element_type=jnp.float32)
    t_ref[1] += lax.dot_general(x, xs1, dn, preferred_element_type=jnp.float32)
    t_ref[2] += lax.dot_general(x, xs2, dn, preferred_element_type=jnp.float32)
    t_ref[3] += lax.dot_general(xs1, xs2, dn, preferred_element_type=jnp.float32)


def _tc_moments(domain2d, feats):
    return pl.pallas_call(
        _mm_body,
        grid=(_B // _BLK,),
        in_specs=[
            pl.BlockSpec((_BLK, 1), lambda i: (i, 0)),
            pl.BlockSpec((_BLK, _D), lambda i: (i, 0)),
        ],
        out_specs=pl.BlockSpec((_M, _D, _D), lambda i: (0, 0, 0)),
        out_shape=jax.ShapeDtypeStruct((_M, _D, _D), jnp.float32),
        compiler_params=pltpu.CompilerParams(dimension_semantics=("arbitrary",)),
    )(domain2d, feats)


def _outer(v):
    return lax.dot_general(v, v, (((0,), (0,)), ((), ())),
                           preferred_element_type=jnp.float32)


def _ep_body(sums_ref, cnts_ref, t_ref, out_ref):
    sums = (sums_ref[0] + sums_ref[1]).reshape(_M, _CP, _D)
    cnts = (cnts_ref[...].reshape(_NW, _NSEGP // 128, 128).sum(axis=0)
            .reshape(_M, _CP // 128, 128).reshape(_M, _CP))

    csafe = jnp.maximum(cnts, 1.0)
    mu = sums / csafe[:, :, None]
    presf = (cnts > 0.0).astype(jnp.float32)

    anchors = (1.0 - _MOM) * mu * presf[:, :, None]
    ag = jnp.zeros((_CP, _D), jnp.float32)
    for d in range(_M):
        upd = _MOM * ag + (1.0 - _MOM) * mu[d]
        pd = presf[d][:, None]
        ag = pd * upd + (1.0 - pd) * ag
    per = jnp.sum((anchors - ag[None]) ** 2, axis=-1) / _D
    nvalid = jnp.sum(presf)
    caa = jnp.where(nvalid > 0,
                    jnp.sum(per * presf) / jnp.maximum(nvalid, 1.0),
                    0.0)

    t = t_ref[...]
    s2 = [(t[0] + sg1 * t[1] + sg2 * t[2] + sg1 * sg2 * t[3]) * 0.25
          for d in range(_M)
          for sg1, sg2 in [(2 * (d & 1) - 1, 2 * (d >> 1) - 1)]]
    s2_tot = t[0]

    tot = jnp.sum(sums, axis=(0, 1)).reshape(1, _D)
    mu_g = tot / _B
    cov = (s2_tot - _B * _outer(mu_g)) / (_B + 1e-6)
    rows = lax.broadcasted_iota(jnp.int32, (_D, _D), 0)
    cols = lax.broadcasted_iota(jnp.int32, (_D, _D), 1)
    eye = (rows == cols).astype(jnp.float32)
    g_mean = (1.0 - _MOM) * mu_g
    g_cov = _MOM * eye + (1.0 - _MOM) * cov

    loss = jnp.float32(0.0)
    nval = jnp.float32(0.0)
    for d in range(_M):
        cnt = jnp.sum(cnts[d])
        s_row = jnp.sum(sums[d], axis=0).reshape(1, _D)
        mu_d = s_row / jnp.maximum(cnt, 1.0)
        cov_d = (s2[d] - cnt * _outer(mu_d)) / (cnt + 1e-6)
        l_d = jnp.mean((mu_d - g_mean) ** 2) + jnp.mean((cov_d - g_cov) ** 2)
        has = (cnt > 0).astype(jnp.float32)
        loss = loss + has * l_d
        nval = nval + has
    stats = jnp.where(nval > 0, loss / jnp.maximum(nval, 1.0), 0.0)

    out_ref[...] = jnp.full((1, 1), caa + stats, jnp.float32)


def _tc_epilogue(sums_p, cnts_p, s2):
    return pl.pallas_call(
        _ep_body,
        out_shape=jax.ShapeDtypeStruct((1, 1), jnp.float32),
    )(sums_p, cnts_p, s2)


def kernel(feats, labels, domain_ids):
    s2 = _tc_moments(domain_ids.reshape(_B, 1), feats)
    sums_p, cnts_p = _get_sc_segsum()(feats.reshape(_B * _H, 128),
                                      labels, domain_ids)
    loss = _tc_epilogue(sums_p,
                        cnts_p.reshape(_NW * _NSEGP // 128, 128),
                        s2)
    return loss.reshape(())


# speedup vs baseline: 1.3449x; 1.0148x over previous
import functools

import jax
import jax.numpy as jnp
from jax import lax
from jax.experimental import pallas as pl
from jax.experimental.pallas import tpu as pltpu
from jax.experimental.pallas import tpu_sc as plsc

_C = 1000
_D = 256
_M = 4
_MOM = 0.9
_B = 16384
_CP = 1024
_NSEGP = _M * _CP
_H = _D // 128

_NC = 2
_NS = 16
_L = 16
_NW = _NC * _NS
_RPT = _B // _NW
_CHUNK = 64
_NCHUNK = _RPT // _CHUNK
_SEG_PT = _NSEGP // _NS


def _sc_body(feats_hbm, labels_hbm, domains_hbm, out_sums, out_cnts,
             fb0, fb1, idx2d, lab_v, dom_v, seg_v, cnt_v, acc_sh, sem0, sem1):
    cid = lax.axis_index("c")
    sid = lax.axis_index("s")
    wid = sid * _NC + cid
    base = wid * _RPT

    cp0 = pltpu.async_copy(
        feats_hbm.at[pl.ds(_H * base, _H * _CHUNK)], fb1, sem1)

    pltpu.sync_copy(labels_hbm.at[pl.ds(base, _RPT)], lab_v)
    pltpu.sync_copy(domains_hbm.at[pl.ds(base, _RPT)], dom_v)

    zrow = jnp.zeros((_L,), jnp.float32)

    def _fill_row(i, carry):
        for k in range(128 // _L):
            fb0[i, pl.ds(k * _L, _L)] = zrow
        cnt_v[pl.ds(i * _L, _L)] = zrow
        cnt_v[pl.ds((i + 128) * _L, _L)] = zrow
        return carry

    lax.fori_loop(0, _H * _CHUNK, _fill_row, 0)

    r0 = sid * _SEG_PT * _H
    for q in range(_SEG_PT * _H // (_H * _CHUNK)):
        pltpu.sync_copy(fb0, acc_sh.at[pl.ds(r0 + q * _H * _CHUNK, _H * _CHUNK)])

    onesv = zrow + 1.0
    for i in range(_RPT // _L):
        seg = dom_v[pl.ds(i * _L, _L)] * _CP + lab_v[pl.ds(i * _L, _L)]
        seg_v[pl.ds(i * _L, _L)] = seg
        plsc.addupdate_scatter(cnt_v, [seg], onesv)

    iota = lax.broadcasted_iota(jnp.int32, (_L,), 0)
    half = iota & 1
    rofs = iota >> 1
    for k in range(_NCHUNK):
        for g8 in range(8):
            gg = k * 8 + g8
            segs = plsc.load_gather(seg_v, [gg * 8 + rofs])
            idx2d[k, pl.ds(g8 * _L, _L)] = segs * _H + half

    plsc.subcore_barrier()

    descs = [cp0]
    for j in range(_NCHUNK):
        descs[j].wait()
        if j + 1 < _NCHUNK:
            nref = fb0 if (j + 1) % 2 == 1 else fb1
            sem = sem0 if (j + 1) % 2 == 1 else sem1
            descs.append(pltpu.async_copy(
                feats_hbm.at[pl.ds(_H * (base + (j + 1) * _CHUNK), _H * _CHUNK)],
                nref, sem))
        cref = fb1 if j % 2 == 0 else fb0
        pltpu.sync_copy(cref, acc_sh.at[idx2d.at[j]], add=True)

    plsc.subcore_barrier()

    pltpu.sync_copy(acc_sh.at[pl.ds(r0, _SEG_PT * _H)],
                    out_sums.at[cid, pl.ds(r0, _SEG_PT * _H)])
    pltpu.sync_copy(cnt_v, out_cnts.at[wid])


@functools.cache
def _get_sc_segsum():
    return pl.kernel(
        _sc_body,
        out_type=(
            jax.ShapeDtypeStruct((_NC, _NSEGP * _H, 128), jnp.float32),
            jax.ShapeDtypeStruct((_NW, _NSEGP), jnp.float32),
        ),
        mesh=plsc.VectorSubcoreMesh(core_axis_name="c", subcore_axis_name="s"),
        compiler_params=pltpu.CompilerParams(use_tc_tiling_on_sc=False,
                                             needs_layout_passes=False),
        scratch_types=[
            pltpu.VMEM((_H * _CHUNK, 128), jnp.float32),
            pltpu.VMEM((_H * _CHUNK, 128), jnp.float32),
            pltpu.VMEM((_NCHUNK, 128), jnp.int32),
            pltpu.VMEM((_RPT,), jnp.int32),
            pltpu.VMEM((_RPT,), jnp.int32),
            pltpu.VMEM((_RPT,), jnp.int32),
            pltpu.VMEM((_NSEGP,), jnp.float32),
            pltpu.VMEM_SHARED((_NSEGP * _H, 128), jnp.float32),
            pltpu.SemaphoreType.DMA,
            pltpu.SemaphoreType.DMA,
        ],
    )


_BLK = 1024


def _mm_body(dom_ref, x_ref, t_ref):
    i = pl.program_id(0)

    @pl.when(i == 0)
    def _init():
        t_ref[...] = jnp.zeros_like(t_ref)

    x = x_ref[...].astype(jnp.bfloat16)
    dom = dom_ref[...]
    xs1 = jnp.where((dom & 1) == 1, x, -x)
    xs2 = jnp.where((dom & 2) == 2, x, -x)
    dn = (((0,), (0,)), ((), ()))
    t_ref[0] += lax.dot_general(x, x, dn, preferred_element_type=jnp.float32)
    t_ref[1] += lax.dot_general(x, xs1, dn, preferred_element_type=jnp.float32)
    t_ref[2] += lax.dot_general(x, xs2, dn, preferred_element_type=jnp.float32)
    t_ref[3] += lax.dot_general(xs1, xs2, dn, preferred_element_type=jnp.float32)


def _tc_moments(domain2d, feats):
    return pl.pallas_call(
        _mm_body,
        grid=(_B // _BLK,),
        in_specs=[
            pl.BlockSpec((_BLK, 1), lambda i: (i, 0)),
            pl.BlockSpec((_BLK, _D), lambda i: (i, 0)),
        ],
        out_specs=pl.BlockSpec((_M, _D, _D), lambda i: (0, 0, 0)),
        out_shape=jax.ShapeDtypeStruct((_M, _D, _D), jnp.float32),
        compiler_params=pltpu.CompilerParams(dimension_semantics=("arbitrary",)),
    )(domain2d, feats)


def _outer(v):
    return lax.dot_general(v, v, (((0,), (0,)), ((), ())),
                           preferred_element_type=jnp.float32)


def _ep_body(sums_ref, cnts_ref, t_ref, out_ref):
    sums = (sums_ref[0] + sums_ref[1]).reshape(_M, _CP, _D)
    cnts = (cnts_ref[...].reshape(_NW, _NSEGP // 128, 128).sum(axis=0)
            .reshape(_M, _CP // 128, 128).reshape(_M, _CP))

    csafe = jnp.maximum(cnts, 1.0)
    mu = sums / csafe[:, :, None]
    presf = (cnts > 0.0).astype(jnp.float32)

    anchors = (1.0 - _MOM) * mu * presf[:, :, None]
    ag = jnp.zeros((_CP, _D), jnp.float32)
    for d in range(_M):
        upd = _MOM * ag + (1.0 - _MOM) * mu[d]
        pd = presf[d][:, None]
        ag = pd * upd + (1.0 - pd) * ag
    per = jnp.sum((anchors - ag[None]) ** 2, axis=-1) / _D
    nvalid = jnp.sum(presf)
    caa = jnp.where(nvalid > 0,
                    jnp.sum(per * presf) / jnp.maximum(nvalid, 1.0),
                    0.0)

    t = t_ref[...]
    s2 = [(t[0] + sg1 * t[1] + sg2 * t[2] + sg1 * sg2 * t[3]) * 0.25
          for d in range(_M)
          for sg1, sg2 in [(2 * (d & 1) - 1, 2 * (d >> 1) - 1)]]
    s2_tot = t[0]

    tot = jnp.sum(sums, axis=(0, 1)).reshape(1, _D)
    mu_g = tot / _B
    cov = (s2_tot - _B * _outer(mu_g)) / (_B + 1e-6)
    rows = lax.broadcasted_iota(jnp.int32, (_D, _D), 0)
    cols = lax.broadcasted_iota(jnp.int32, (_D, _D), 1)
    eye = (rows == cols).astype(jnp.float32)
    g_mean = (1.0 - _MOM) * mu_g
    g_cov = _MOM * eye + (1.0 - _MOM) * cov

    loss = jnp.float32(0.0)
    nval = jnp.float32(0.0)
    for d in range(_M):
        cnt = jnp.sum(cnts[d])
        s_row = jnp.sum(sums[d], axis=0).reshape(1, _D)
        mu_d = s_row / jnp.maximum(cnt, 1.0)
        cov_d = (s2[d] - cnt * _outer(mu_d)) / (cnt + 1e-6)
        l_d = jnp.mean((mu_d - g_mean) ** 2) + jnp.mean((cov_d - g_cov) ** 2)
        has = (cnt > 0).astype(jnp.float32)
        loss = loss + has * l_d
        nval = nval + has
    stats = jnp.where(nval > 0, loss / jnp.maximum(nval, 1.0), 0.0)

    out_ref[...] = jnp.full((1, 1), caa + stats, jnp.float32)


def _tc_epilogue(sums_p, cnts_p, s2):
    return pl.pallas_call(
        _ep_body,
        out_shape=jax.ShapeDtypeStruct((1, 1), jnp.float32),
    )(sums_p, cnts_p, s2)


def kernel(feats, labels, domain_ids):
    s2 = _tc_moments(domain_ids.reshape(_B, 1), feats)
    sums_p, cnts_p = _get_sc_segsum()(feats.reshape(_B * _H, 128),
                                      labels, domain_ids)
    loss = _tc_epilogue(sums_p,
                        cnts_p.reshape(_NW * _NSEGP // 128, 128),
                        s2)
    return loss.reshape(())


# speedup vs baseline: 1.7475x; 1.2993x over previous
import functools

import jax
import jax.numpy as jnp
from jax import lax
from jax.experimental import pallas as pl
from jax.experimental.pallas import tpu as pltpu
from jax.experimental.pallas import tpu_sc as plsc

_C = 1000
_D = 256
_M = 4
_MOM = 0.9
_B = 16384
_CP = 1024
_NSEGP = _M * _CP
_H = _D // 128

_NC = 2
_NS = 16
_L = 16
_NW = _NC * _NS
_RPT = _B // _NW
_CHUNK = 64
_NCHUNK = _RPT // _CHUNK
_SEG_PT = _NSEGP // _NS


def _sc_body(feats_hbm, labels_hbm, domains_hbm, out_sums, out_cnts,
             fb0, fb1, idx2d, lab_v, dom_v, seg_v, cnt_v, acc_sh, sem0, sem1):
    cid = lax.axis_index("c")
    sid = lax.axis_index("s")
    wid = sid * _NC + cid
    base = wid * _RPT

    cp0 = pltpu.async_copy(
        feats_hbm.at[pl.ds(_H * base, _H * _CHUNK)], fb1, sem1)

    pltpu.sync_copy(labels_hbm.at[pl.ds(base, _RPT)], lab_v)
    pltpu.sync_copy(domains_hbm.at[pl.ds(base, _RPT)], dom_v)

    zrow = jnp.zeros((_L,), jnp.float32)

    def _fill_row(i, carry):
        for k in range(128 // _L):
            fb0[i, pl.ds(k * _L, _L)] = zrow
        cnt_v[pl.ds(i * _L, _L)] = zrow
        cnt_v[pl.ds((i + 128) * _L, _L)] = zrow
        return carry

    lax.fori_loop(0, _H * _CHUNK, _fill_row, 0)

    r0 = sid * _SEG_PT * _H
    for q in range(_SEG_PT * _H // (_H * _CHUNK)):
        pltpu.sync_copy(fb0, acc_sh.at[pl.ds(r0 + q * _H * _CHUNK, _H * _CHUNK)])

    onesv = zrow + 1.0
    for i in range(_RPT // _L):
        seg = dom_v[pl.ds(i * _L, _L)] * _CP + lab_v[pl.ds(i * _L, _L)]
        seg_v[pl.ds(i * _L, _L)] = seg
        plsc.addupdate_scatter(cnt_v, [seg], onesv)

    iota = lax.broadcasted_iota(jnp.int32, (_L,), 0)
    half = iota >> 3
    rofs = iota & 7
    for k in range(_NCHUNK):
        for g8 in range(8):
            gg = k * 8 + g8
            segs = plsc.load_gather(seg_v, [gg * 8 + rofs])
            idx2d[k, pl.ds(g8 * _L, _L)] = segs * _H + half

    plsc.subcore_barrier()

    descs = [cp0]
    for j in range(_NCHUNK):
        descs[j].wait()
        if j + 1 < _NCHUNK:
            nref = fb0 if (j + 1) % 2 == 1 else fb1
            sem = sem0 if (j + 1) % 2 == 1 else sem1
            descs.append(pltpu.async_copy(
                feats_hbm.at[pl.ds(_H * (base + (j + 1) * _CHUNK), _H * _CHUNK)],
                nref, sem))
        cref = fb1 if j % 2 == 0 else fb0
        pltpu.sync_copy(cref, acc_sh.at[idx2d.at[j]], add=True)

    plsc.subcore_barrier()

    pltpu.sync_copy(acc_sh.at[pl.ds(r0, _SEG_PT * _H)],
                    out_sums.at[cid, pl.ds(r0, _SEG_PT * _H)])
    pltpu.sync_copy(cnt_v, out_cnts.at[wid])


@functools.cache
def _get_sc_segsum():
    return pl.kernel(
        _sc_body,
        out_type=(
            jax.ShapeDtypeStruct((_NC, _NSEGP * _H, 128), jnp.float32),
            jax.ShapeDtypeStruct((_NW, _NSEGP), jnp.float32),
        ),
        mesh=plsc.VectorSubcoreMesh(core_axis_name="c", subcore_axis_name="s"),
        compiler_params=pltpu.CompilerParams(use_tc_tiling_on_sc=False,
                                             needs_layout_passes=False),
        scratch_types=[
            pltpu.VMEM((_H * _CHUNK, 128), jnp.float32),
            pltpu.VMEM((_H * _CHUNK, 128), jnp.float32),
            pltpu.VMEM((_NCHUNK, 128), jnp.int32),
            pltpu.VMEM((_RPT,), jnp.int32),
            pltpu.VMEM((_RPT,), jnp.int32),
            pltpu.VMEM((_RPT,), jnp.int32),
            pltpu.VMEM((_NSEGP,), jnp.float32),
            pltpu.VMEM_SHARED((_NSEGP * _H, 128), jnp.float32),
            pltpu.SemaphoreType.DMA,
            pltpu.SemaphoreType.DMA,
        ],
    )


_BLK = 1024


def _mm_body(dom_ref, x_ref, t_ref):
    i = pl.program_id(0)

    @pl.when(i == 0)
    def _init():
        t_ref[...] = jnp.zeros_like(t_ref)

    x = x_ref[...].astype(jnp.bfloat16)
    dom = dom_ref[...]
    xs1 = jnp.where((dom & 1) == 1, x, -x)
    xs2 = jnp.where((dom & 2) == 2, x, -x)
    dn = (((0,), (0,)), ((), ()))
    t_ref[0] += lax.dot_general(x, x, dn, preferred_element_type=jnp.float32)
    t_ref[1] += lax.dot_general(x, xs1, dn, preferred_element_type=jnp.float32)
    t_ref[2] += lax.dot_general(x, xs2, dn, preferred_element_type=jnp.float32)
    t_ref[3] += lax.dot_general(xs1, xs2, dn, preferred_element_type=jnp.float32)


def _tc_moments(domain2d, feats):
    return pl.pallas_call(
        _mm_body,
        grid=(_B // _BLK,),
        in_specs=[
            pl.BlockSpec((_BLK, 1), lambda i: (i, 0)),
            pl.BlockSpec((_BLK, _D), lambda i: (i, 0)),
        ],
        out_specs=pl.BlockSpec((_M, _D, _D), lambda i: (0, 0, 0)),
        out_shape=jax.ShapeDtypeStruct((_M, _D, _D), jnp.float32),
        compiler_params=pltpu.CompilerParams(dimension_semantics=("arbitrary",)),
    )(domain2d, feats)


def _outer(v):
    return lax.dot_general(v, v, (((0,), (0,)), ((), ())),
                           preferred_element_type=jnp.float32)


def _ep_body(sums_ref, cnts_ref, t_ref, out_ref):
    sums = (sums_ref[0] + sums_ref[1]).reshape(_M, _CP, _D)
    cnts = (cnts_ref[...].reshape(_NW, _NSEGP // 128, 128).sum(axis=0)
            .reshape(_M, _CP // 128, 128).reshape(_M, _CP))

    csafe = jnp.maximum(cnts, 1.0)
    mu = sums / csafe[:, :, None]
    presf = (cnts > 0.0).astype(jnp.float32)

    anchors = (1.0 - _MOM) * mu * presf[:, :, None]
    ag = jnp.zeros((_CP, _D), jnp.float32)
    for d in range(_M):
        upd = _MOM * ag + (1.0 - _MOM) * mu[d]
        pd = presf[d][:, None]
        ag = pd * upd + (1.0 - pd) * ag
    per = jnp.sum((anchors - ag[None]) ** 2, axis=-1) / _D
    nvalid = jnp.sum(presf)
    caa = jnp.where(nvalid > 0,
                    jnp.sum(per * presf) / jnp.maximum(nvalid, 1.0),
                    0.0)

    t = t_ref[...]
    s2 = [(t[0] + sg1 * t[1] + sg2 * t[2] + sg1 * sg2 * t[3]) * 0.25
          for d in range(_M)
          for sg1, sg2 in [(2 * (d & 1) - 1, 2 * (d >> 1) - 1)]]
    s2_tot = t[0]

    tot = jnp.sum(sums, axis=(0, 1)).reshape(1, _D)
    mu_g = tot / _B
    cov = (s2_tot - _B * _outer(mu_g)) / (_B + 1e-6)
    rows = lax.broadcasted_iota(jnp.int32, (_D, _D), 0)
    cols = lax.broadcasted_iota(jnp.int32, (_D, _D), 1)
    eye = (rows == cols).astype(jnp.float32)
    g_mean = (1.0 - _MOM) * mu_g
    g_cov = _MOM * eye + (1.0 - _MOM) * cov

    loss = jnp.float32(0.0)
    nval = jnp.float32(0.0)
    for d in range(_M):
        cnt = jnp.sum(cnts[d])
        s_row = jnp.sum(sums[d], axis=0).reshape(1, _D)
        mu_d = s_row / jnp.maximum(cnt, 1.0)
        cov_d = (s2[d] - cnt * _outer(mu_d)) / (cnt + 1e-6)
        l_d = jnp.mean((mu_d - g_mean) ** 2) + jnp.mean((cov_d - g_cov) ** 2)
        has = (cnt > 0).astype(jnp.float32)
        loss = loss + has * l_d
        nval = nval + has
    stats = jnp.where(nval > 0, loss / jnp.maximum(nval, 1.0), 0.0)

    out_ref[...] = jnp.full((1, 1), caa + stats, jnp.float32)


def _tc_epilogue(sums_p, cnts_p, s2):
    return pl.pallas_call(
        _ep_body,
        out_shape=jax.ShapeDtypeStruct((1, 1), jnp.float32),
    )(sums_p, cnts_p, s2)


def kernel(feats, labels, domain_ids):
    s2 = _tc_moments(domain_ids.reshape(_B, 1), feats)
    feats_t = (feats.reshape(_B // 8, 8, _H, 128)
               .transpose(0, 2, 1, 3).reshape(_B * _H, 128))
    sums_p, cnts_p = _get_sc_segsum()(feats_t, labels, domain_ids)
    loss = _tc_epilogue(sums_p,
                        cnts_p.reshape(_NW * _NSEGP // 128, 128),
                        s2)
    return loss.reshape(())


# speedup vs baseline: 1.8041x; 1.0324x over previous
import functools

import jax
import jax.numpy as jnp
from jax import lax
from jax.experimental import pallas as pl
from jax.experimental.pallas import tpu as pltpu
from jax.experimental.pallas import tpu_sc as plsc

_C = 1000
_D = 256
_M = 4
_MOM = 0.9
_B = 16384
_CP = 1024
_NSEGP = _M * _CP
_H = _D // 128

_NC = 2
_NS = 16
_L = 16
_NW = _NC * _NS
_RPT = _B // _NW
_CHUNK = 64
_NCHUNK = _RPT // _CHUNK
_SEG_PT = _NSEGP // _NS


def _sc_body(feats_hbm, labels_hbm, domains_hbm, out_sums, out_cnts,
             fb0, fb1, idx2d, lab_v, dom_v, seg_v, cnt_v, acc_sh, sem0, sem1):
    cid = lax.axis_index("c")
    sid = lax.axis_index("s")
    wid = sid * _NC + cid
    base = wid * _RPT

    cp0 = pltpu.async_copy(
        feats_hbm.at[pl.ds(_H * base, _H * _CHUNK)], fb1, sem1)

    pltpu.sync_copy(labels_hbm.at[pl.ds(base, _RPT)], lab_v)
    pltpu.sync_copy(domains_hbm.at[pl.ds(base, _RPT)], dom_v)

    zrow = jnp.zeros((_L,), jnp.float32)

    def _fill_row(i, carry):
        for k in range(128 // _L):
            fb0[i, pl.ds(k * _L, _L)] = zrow
        cnt_v[pl.ds(i * _L, _L)] = zrow
        cnt_v[pl.ds((i + 128) * _L, _L)] = zrow
        return carry

    lax.fori_loop(0, _H * _CHUNK, _fill_row, 0)

    r0 = sid * _SEG_PT * _H
    for q in range(_SEG_PT * _H // (_H * _CHUNK)):
        pltpu.sync_copy(fb0, acc_sh.at[pl.ds(r0 + q * _H * _CHUNK, _H * _CHUNK)])

    onesv = zrow + 1.0
    for i in range(_RPT // _L):
        seg = dom_v[pl.ds(i * _L, _L)] * _CP + lab_v[pl.ds(i * _L, _L)]
        seg_v[pl.ds(i * _L, _L)] = seg
        plsc.addupdate_scatter(cnt_v, [seg], onesv)

    iota = lax.broadcasted_iota(jnp.int32, (_L,), 0)
    half = iota >> 3
    rofs = iota & 7
    for k in range(_NCHUNK):
        for g8 in range(8):
            gg = k * 8 + g8
            segs = plsc.load_gather(seg_v, [gg * 8 + rofs])
            idx2d[k, pl.ds(g8 * _L, _L)] = segs * _H + half

    plsc.subcore_barrier()

    descs = [cp0]
    for j in range(_NCHUNK):
        descs[j].wait()
        if j + 1 < _NCHUNK:
            nref = fb0 if (j + 1) % 2 == 1 else fb1
            sem = sem0 if (j + 1) % 2 == 1 else sem1
            descs.append(pltpu.async_copy(
                feats_hbm.at[pl.ds(_H * (base + (j + 1) * _CHUNK), _H * _CHUNK)],
                nref, sem))
        cref = fb1 if j % 2 == 0 else fb0
        pltpu.sync_copy(cref, acc_sh.at[idx2d.at[j]], add=True)

    plsc.subcore_barrier()

    pltpu.sync_copy(acc_sh.at[pl.ds(r0, _SEG_PT * _H)],
                    out_sums.at[cid, pl.ds(r0, _SEG_PT * _H)])
    pltpu.sync_copy(cnt_v, out_cnts.at[wid])


@functools.cache
def _get_sc_segsum():
    return pl.kernel(
        _sc_body,
        out_type=(
            jax.ShapeDtypeStruct((_NC, _NSEGP * _H, 128), jnp.float32),
            jax.ShapeDtypeStruct((_NW, _NSEGP), jnp.float32),
        ),
        mesh=plsc.VectorSubcoreMesh(core_axis_name="c", subcore_axis_name="s"),
        compiler_params=pltpu.CompilerParams(use_tc_tiling_on_sc=False,
                                             needs_layout_passes=False),
        scratch_types=[
            pltpu.VMEM((_H * _CHUNK, 128), jnp.float32),
            pltpu.VMEM((_H * _CHUNK, 128), jnp.float32),
            pltpu.VMEM((_NCHUNK, 128), jnp.int32),
            pltpu.VMEM((_RPT,), jnp.int32),
            pltpu.VMEM((_RPT,), jnp.int32),
            pltpu.VMEM((_RPT,), jnp.int32),
            pltpu.VMEM((_NSEGP,), jnp.float32),
            pltpu.VMEM_SHARED((_NSEGP * _H, 128), jnp.float32),
            pltpu.SemaphoreType.DMA,
            pltpu.SemaphoreType.DMA,
        ],
    )


_BLK = 1024


def _mm_body(dom_ref, x_ref, t_ref):
    i = pl.program_id(0)

    @pl.when(i == 0)
    def _init():
        t_ref[...] = jnp.zeros_like(t_ref)

    x = x_ref[...].astype(jnp.bfloat16)
    dom = dom_ref[...].astype(jnp.int32)
    xs1 = jnp.where((dom & 1) == 1, x, -x)
    xs2 = jnp.where((dom & 2) == 2, x, -x)
    dn = (((0,), (0,)), ((), ()))
    t_ref[0] += lax.dot_general(x, x, dn, preferred_element_type=jnp.float32)
    t_ref[1] += lax.dot_general(x, xs1, dn, preferred_element_type=jnp.float32)
    t_ref[2] += lax.dot_general(x, xs2, dn, preferred_element_type=jnp.float32)
    t_ref[3] += lax.dot_general(xs1, xs2, dn, preferred_element_type=jnp.float32)


def _tc_moments(domain2d, feats):
    return pl.pallas_call(
        _mm_body,
        grid=(_B // _BLK,),
        in_specs=[
            pl.BlockSpec((_BLK, 1), lambda i: (i, 0)),
            pl.BlockSpec((_BLK, _D), lambda i: (i, 0)),
        ],
        out_specs=pl.BlockSpec((_M, _D, _D), lambda i: (0, 0, 0)),
        out_shape=jax.ShapeDtypeStruct((_M, _D, _D), jnp.float32),
        compiler_params=pltpu.CompilerParams(dimension_semantics=("arbitrary",)),
    )(domain2d, feats)


def _outer(v):
    return lax.dot_general(v, v, (((0,), (0,)), ((), ())),
                           preferred_element_type=jnp.float32)


def _ep_body(sums_ref, cnts_ref, t_ref, out_ref):
    sums = (sums_ref[0] + sums_ref[1]).reshape(_M, _CP, _D)
    cnts = (cnts_ref[...].reshape(_NW, _NSEGP // 128, 128).sum(axis=0)
            .reshape(_M, _CP // 128, 128).reshape(_M, _CP))

    csafe = jnp.maximum(cnts, 1.0)
    mu = sums / csafe[:, :, None]
    presf = (cnts > 0.0).astype(jnp.float32)

    anchors = (1.0 - _MOM) * mu * presf[:, :, None]
    ag = jnp.zeros((_CP, _D), jnp.float32)
    for d in range(_M):
        upd = _MOM * ag + (1.0 - _MOM) * mu[d]
        pd = presf[d][:, None]
        ag = pd * upd + (1.0 - pd) * ag
    per = jnp.sum((anchors - ag[None]) ** 2, axis=-1) / _D
    nvalid = jnp.sum(presf)
    caa = jnp.where(nvalid > 0,
                    jnp.sum(per * presf) / jnp.maximum(nvalid, 1.0),
                    0.0)

    t = t_ref[...]
    s2 = [(t[0] + sg1 * t[1] + sg2 * t[2] + sg1 * sg2 * t[3]) * 0.25
          for d in range(_M)
          for sg1, sg2 in [(2 * (d & 1) - 1, 2 * (d >> 1) - 1)]]
    s2_tot = t[0]

    tot = jnp.sum(sums, axis=(0, 1)).reshape(1, _D)
    mu_g = tot / _B
    cov = (s2_tot - _B * _outer(mu_g)) / (_B + 1e-6)
    rows = lax.broadcasted_iota(jnp.int32, (_D, _D), 0)
    cols = lax.broadcasted_iota(jnp.int32, (_D, _D), 1)
    eye = (rows == cols).astype(jnp.float32)
    g_mean = (1.0 - _MOM) * mu_g
    g_cov = _MOM * eye + (1.0 - _MOM) * cov

    loss = jnp.float32(0.0)
    nval = jnp.float32(0.0)
    for d in range(_M):
        cnt = jnp.sum(cnts[d])
        s_row = jnp.sum(sums[d], axis=0).reshape(1, _D)
        mu_d = s_row / jnp.maximum(cnt, 1.0)
        cov_d = (s2[d] - cnt * _outer(mu_d)) / (cnt + 1e-6)
        l_d = jnp.mean((mu_d - g_mean) ** 2) + jnp.mean((cov_d - g_cov) ** 2)
        has = (cnt > 0).astype(jnp.float32)
        loss = loss + has * l_d
        nval = nval + has
    stats = jnp.where(nval > 0, loss / jnp.maximum(nval, 1.0), 0.0)

    out_ref[...] = jnp.full((1, 1), caa + stats, jnp.float32)


def _tc_epilogue(sums_p, cnts_p, s2):
    return pl.pallas_call(
        _ep_body,
        out_shape=jax.ShapeDtypeStruct((1, 1), jnp.float32),
    )(sums_p, cnts_p, s2)


def kernel(feats, labels, domain_ids):
    s2 = _tc_moments(domain_ids.astype(jnp.int8).reshape(_B, 1), feats)
    feats_t = (feats.reshape(_B // 8, 8, _H, 128)
               .transpose(0, 2, 1, 3).reshape(_B * _H, 128))
    sums_p, cnts_p = _get_sc_segsum()(feats_t, labels, domain_ids)
    loss = _tc_epilogue(sums_p,
                        cnts_p.reshape(_NW * _NSEGP // 128, 128),
                        s2)
    return loss.reshape(())


# speedup vs baseline: 1.8887x; 1.0469x over previous
import functools

import jax
import jax.numpy as jnp
from jax import lax
from jax.experimental import pallas as pl
from jax.experimental.pallas import tpu as pltpu
from jax.experimental.pallas import tpu_sc as plsc

_C = 1000
_D = 256
_M = 4
_MOM = 0.9
_B = 16384
_CP = 1024
_NSEGP = _M * _CP
_H = _D // 128

_NC = 2
_NS = 16
_L = 16
_NW = _NC * _NS
_RPT2 = _B // _NS
_CHUNK2 = 128
_NCHUNK2 = _RPT2 // _CHUNK2


def _sc_body(feats_hbm, labels_hbm, domains_hbm, out_sums, out_cnts,
             buf0, buf1, gidx, seg2d, lab_v, dom_v, cnt_v, acc_sh, sem0, sem1):
    cid = lax.axis_index("c")
    sid = lax.axis_index("s")
    rbase = sid * _RPT2

    iota = lax.broadcasted_iota(jnp.int32, (_L,), 0)
    for k in range(_NCHUNK2):
        for i in range(_CHUNK2 // _L):
            rv = rbase + k * _CHUNK2 + i * _L + iota
            gidx[k, pl.ds(i * _L, _L)] = ((rv >> 3) << 4) + 8 * cid + (rv & 7)

    pltpu.sync_copy(labels_hbm.at[pl.ds(rbase, _RPT2)], lab_v)
    pltpu.sync_copy(domains_hbm.at[pl.ds(rbase, _RPT2)], dom_v)

    zrow = jnp.zeros((_L,), jnp.float32)

    def _fill_row(i, carry):
        for k in range(128 // _L):
            buf0[i, pl.ds(k * _L, _L)] = zrow
        cnt_v[pl.ds(i * _L, _L)] = zrow
        cnt_v[pl.ds((i + 128) * _L, _L)] = zrow
        return carry

    lax.fori_loop(0, _CHUNK2, _fill_row, 0)

    r0 = sid * (_NSEGP // _NS)
    pltpu.sync_copy(buf0, acc_sh.at[pl.ds(r0, _CHUNK2)])
    pltpu.sync_copy(buf0, acc_sh.at[pl.ds(r0 + _CHUNK2, _CHUNK2)])

    onesv = zrow + 1.0
    for i in range(_RPT2 // _L):
        seg = dom_v[pl.ds(i * _L, _L)] * _CP + lab_v[pl.ds(i * _L, _L)]
        seg2d[i // 8, pl.ds((i % 8) * _L, _L)] = seg
        plsc.addupdate_scatter(cnt_v, [seg], onesv)

    cp0 = pltpu.async_copy(feats_hbm.at[gidx.at[0]], buf1, sem1)

    plsc.subcore_barrier()

    descs = [cp0]
    for k in range(_NCHUNK2):
        descs[k].wait()
        if k + 1 < _NCHUNK2:
            nref = buf0 if (k + 1) % 2 == 1 else buf1
            sem = sem0 if (k + 1) % 2 == 1 else sem1
            descs.append(pltpu.async_copy(
                feats_hbm.at[gidx.at[k + 1]], nref, sem))
        cref = buf1 if k % 2 == 0 else buf0
        pltpu.sync_copy(cref, acc_sh.at[seg2d.at[k]], add=True)

    plsc.subcore_barrier()

    pltpu.sync_copy(acc_sh.at[pl.ds(r0, _NSEGP // _NS)],
                    out_sums.at[cid, pl.ds(r0, _NSEGP // _NS)])

    @pl.when(cid == 0)
    def _():
        pltpu.sync_copy(cnt_v, out_cnts.at[sid])


@functools.cache
def _get_sc_segsum():
    return pl.kernel(
        _sc_body,
        out_type=(
            jax.ShapeDtypeStruct((_H, _NSEGP, 128), jnp.float32),
            jax.ShapeDtypeStruct((_NS, _NSEGP), jnp.float32),
        ),
        mesh=plsc.VectorSubcoreMesh(core_axis_name="c", subcore_axis_name="s"),
        compiler_params=pltpu.CompilerParams(use_tc_tiling_on_sc=False,
                                             needs_layout_passes=False),
        scratch_types=[
            pltpu.VMEM((_CHUNK2, 128), jnp.float32),
            pltpu.VMEM((_CHUNK2, 128), jnp.float32),
            pltpu.VMEM((_NCHUNK2, 128), jnp.int32),
            pltpu.VMEM((_NCHUNK2, 128), jnp.int32),
            pltpu.VMEM((_RPT2,), jnp.int32),
            pltpu.VMEM((_RPT2,), jnp.int32),
            pltpu.VMEM((_NSEGP,), jnp.float32),
            pltpu.VMEM_SHARED((_NSEGP, 128), jnp.float32),
            pltpu.SemaphoreType.DMA,
            pltpu.SemaphoreType.DMA,
        ],
    )


_BLK = 1024


def _mm_body(dom_ref, x_ref, t_ref):
    i = pl.program_id(0)

    @pl.when(i == 0)
    def _init():
        t_ref[...] = jnp.zeros_like(t_ref)

    x = x_ref[...].astype(jnp.bfloat16)
    dom = dom_ref[...].astype(jnp.int32)
    xs1 = jnp.where((dom & 1) == 1, x, -x)
    xs2 = jnp.where((dom & 2) == 2, x, -x)
    dn = (((0,), (0,)), ((), ()))
    t_ref[0] += lax.dot_general(x, x, dn, preferred_element_type=jnp.float32)
    t_ref[1] += lax.dot_general(x, xs1, dn, preferred_element_type=jnp.float32)
    t_ref[2] += lax.dot_general(x, xs2, dn, preferred_element_type=jnp.float32)
    t_ref[3] += lax.dot_general(xs1, xs2, dn, preferred_element_type=jnp.float32)


def _tc_moments(domain2d, feats):
    return pl.pallas_call(
        _mm_body,
        grid=(_B // _BLK,),
        in_specs=[
            pl.BlockSpec((_BLK, 1), lambda i: (i, 0)),
            pl.BlockSpec((_BLK, _D), lambda i: (i, 0)),
        ],
        out_specs=pl.BlockSpec((_M, _D, _D), lambda i: (0, 0, 0)),
        out_shape=jax.ShapeDtypeStruct((_M, _D, _D), jnp.float32),
        compiler_params=pltpu.CompilerParams(dimension_semantics=("arbitrary",)),
    )(domain2d, feats)


def _outer(v):
    return lax.dot_general(v, v, (((0,), (0,)), ((), ())),
                           preferred_element_type=jnp.float32)


def _ep_body(sums_ref, cnts_ref, t_ref, out_ref):
    sums = jnp.concatenate([sums_ref[0], sums_ref[1]],
                           axis=-1).reshape(_M, _CP, _D)
    cnts = (cnts_ref[...].reshape(_NS, _NSEGP // 128, 128).sum(axis=0)
            .reshape(_M, _CP // 128, 128).reshape(_M, _CP))

    csafe = jnp.maximum(cnts, 1.0)
    mu = sums / csafe[:, :, None]
    presf = (cnts > 0.0).astype(jnp.float32)

    anchors = (1.0 - _MOM) * mu * presf[:, :, None]
    ag = jnp.zeros((_CP, _D), jnp.float32)
    for d in range(_M):
        upd = _MOM * ag + (1.0 - _MOM) * mu[d]
        pd = presf[d][:, None]
        ag = pd * upd + (1.0 - pd) * ag
    per = jnp.sum((anchors - ag[None]) ** 2, axis=-1) / _D
    nvalid = jnp.sum(presf)
    caa = jnp.where(nvalid > 0,
                    jnp.sum(per * presf) / jnp.maximum(nvalid, 1.0),
                    0.0)

    t = t_ref[...]
    s2 = [(t[0] + sg1 * t[1] + sg2 * t[2] + sg1 * sg2 * t[3]) * 0.25
          for d in range(_M)
          for sg1, sg2 in [(2 * (d & 1) - 1, 2 * (d >> 1) - 1)]]
    s2_tot = t[0]

    tot = jnp.sum(sums, axis=(0, 1)).reshape(1, _D)
    mu_g = tot / _B
    cov = (s2_tot - _B * _outer(mu_g)) / (_B + 1e-6)
    rows = lax.broadcasted_iota(jnp.int32, (_D, _D), 0)
    cols = lax.broadcasted_iota(jnp.int32, (_D, _D), 1)
    eye = (rows == cols).astype(jnp.float32)
    g_mean = (1.0 - _MOM) * mu_g
    g_cov = _MOM * eye + (1.0 - _MOM) * cov

    loss = jnp.float32(0.0)
    nval = jnp.float32(0.0)
    for d in range(_M):
        cnt = jnp.sum(cnts[d])
        s_row = jnp.sum(sums[d], axis=0).reshape(1, _D)
        mu_d = s_row / jnp.maximum(cnt, 1.0)
        cov_d = (s2[d] - cnt * _outer(mu_d)) / (cnt + 1e-6)
        l_d = jnp.mean((mu_d - g_mean) ** 2) + jnp.mean((cov_d - g_cov) ** 2)
        has = (cnt > 0).astype(jnp.float32)
        loss = loss + has * l_d
        nval = nval + has
    stats = jnp.where(nval > 0, loss / jnp.maximum(nval, 1.0), 0.0)

    out_ref[...] = jnp.full((1, 1), caa + stats, jnp.float32)


def _tc_epilogue(sums_p, cnts_p, s2):
    return pl.pallas_call(
        _ep_body,
        out_shape=jax.ShapeDtypeStruct((1, 1), jnp.float32),
    )(sums_p, cnts_p, s2)


def kernel(feats, labels, domain_ids):
    s2 = _tc_moments(domain_ids.astype(jnp.int8).reshape(_B, 1), feats)
    feats_t = (feats.reshape(_B // 8, 8, _H, 128)
               .transpose(0, 2, 1, 3).reshape(_B * _H, 128))
    sums_p, cnts_p = _get_sc_segsum()(feats_t, labels, domain_ids)
    loss = _tc_epilogue(sums_p,
                        cnts_p.reshape(_NS * _NSEGP // 128, 128),
                        s2)
    return loss.reshape(())


# speedup vs baseline: 1.9229x; 1.0181x over previous
import functools

import jax
import jax.numpy as jnp
from jax import lax
from jax.experimental import pallas as pl
from jax.experimental.pallas import tpu as pltpu
from jax.experimental.pallas import tpu_sc as plsc

_C = 1000
_D = 256
_M = 4
_MOM = 0.9
_B = 16384
_CP = 1024
_NSEGP = _M * _CP
_H = _D // 128

_NC = 2
_NS = 16
_L = 16
_NW = _NC * _NS
_RPT2 = _B // _NS
_CHUNK2 = 128
_NCHUNK2 = _RPT2 // _CHUNK2


def _sc_body(feats_hbm, labels_hbm, domains_hbm, out_sums, out_cnts,
             buf0, buf1, gidx, seg2d, lab_v, dom_v, cnt_v, acc_sh, sem0, sem1):
    cid = lax.axis_index("c")
    sid = lax.axis_index("s")
    rbase = sid * _RPT2

    iota = lax.broadcasted_iota(jnp.int32, (_L,), 0)
    for k in range(_NCHUNK2):
        for i in range(_CHUNK2 // _L):
            rv = rbase + k * _CHUNK2 + i * _L + iota
            gidx[k, pl.ds(i * _L, _L)] = ((rv >> 3) << 4) + 8 * cid + (rv & 7)

    pltpu.sync_copy(labels_hbm.at[pl.ds(rbase, _RPT2)], lab_v)
    pltpu.sync_copy(domains_hbm.at[pl.ds(rbase, _RPT2)], dom_v)

    zrow = jnp.zeros((_L,), jnp.float32)

    def _fill_row(i, carry):
        for k in range(128 // _L):
            buf0[i, pl.ds(k * _L, _L)] = zrow
        cnt_v[pl.ds(i * _L, _L)] = zrow
        cnt_v[pl.ds((i + 128) * _L, _L)] = zrow
        return carry

    lax.fori_loop(0, _CHUNK2, _fill_row, 0)

    r0 = sid * (_NSEGP // _NS)
    pltpu.sync_copy(buf0, acc_sh.at[pl.ds(r0, _CHUNK2)])
    pltpu.sync_copy(buf0, acc_sh.at[pl.ds(r0 + _CHUNK2, _CHUNK2)])

    onesv = zrow + 1.0
    for i in range(_RPT2 // _L):
        seg = dom_v[pl.ds(i * _L, _L)] * _CP + lab_v[pl.ds(i * _L, _L)]
        seg2d[i // 8, pl.ds((i % 8) * _L, _L)] = seg
        plsc.addupdate_scatter(cnt_v, [seg], onesv)

    cp0 = pltpu.async_copy(feats_hbm.at[gidx.at[0]], buf1, sem1)

    plsc.subcore_barrier()

    descs = [cp0]
    for k in range(_NCHUNK2):
        descs[k].wait()
        if k + 1 < _NCHUNK2:
            nref = buf0 if (k + 1) % 2 == 1 else buf1
            sem = sem0 if (k + 1) % 2 == 1 else sem1
            descs.append(pltpu.async_copy(
                feats_hbm.at[gidx.at[k + 1]], nref, sem))
        cref = buf1 if k % 2 == 0 else buf0
        pltpu.sync_copy(cref, acc_sh.at[seg2d.at[k]], add=True)

    plsc.subcore_barrier()

    pltpu.sync_copy(acc_sh.at[pl.ds(r0, _NSEGP // _NS)],
                    out_sums.at[cid, pl.ds(r0, _NSEGP // _NS)])

    @pl.when(cid == 0)
    def _():
        pltpu.sync_copy(cnt_v, out_cnts.at[sid])


@functools.cache
def _get_sc_segsum():
    return pl.kernel(
        _sc_body,
        out_type=(
            jax.ShapeDtypeStruct((_H, _NSEGP, 128), jnp.float32),
            jax.ShapeDtypeStruct((_NS, _NSEGP), jnp.float32),
        ),
        mesh=plsc.VectorSubcoreMesh(core_axis_name="c", subcore_axis_name="s"),
        compiler_params=pltpu.CompilerParams(use_tc_tiling_on_sc=False,
                                             needs_layout_passes=False),
        scratch_types=[
            pltpu.VMEM((_CHUNK2, 128), jnp.float32),
            pltpu.VMEM((_CHUNK2, 128), jnp.float32),
            pltpu.VMEM((_NCHUNK2, 128), jnp.int32),
            pltpu.VMEM((_NCHUNK2, 128), jnp.int32),
            pltpu.VMEM((_RPT2,), jnp.int32),
            pltpu.VMEM((_RPT2,), jnp.int32),
            pltpu.VMEM((_NSEGP,), jnp.float32),
            pltpu.VMEM_SHARED((_NSEGP, 128), jnp.float32),
            pltpu.SemaphoreType.DMA,
            pltpu.SemaphoreType.DMA,
        ],
    )


_BLK = 2048


def _mm_body(dom_ref, x_ref, t_ref):
    i = pl.program_id(0)

    @pl.when(i == 0)
    def _init():
        t_ref[...] = jnp.zeros_like(t_ref)

    x = x_ref[...].astype(jnp.bfloat16)
    dom = dom_ref[...].astype(jnp.int32)
    xs1 = jnp.where((dom & 1) == 1, x, -x)
    xs2 = jnp.where((dom & 2) == 2, x, -x)
    dn = (((0,), (0,)), ((), ()))
    t_ref[0] += lax.dot_general(x, x, dn, preferred_element_type=jnp.float32)
    t_ref[1] += lax.dot_general(x, xs1, dn, preferred_element_type=jnp.float32)
    t_ref[2] += lax.dot_general(x, xs2, dn, preferred_element_type=jnp.float32)
    t_ref[3] += lax.dot_general(xs1, xs2, dn, preferred_element_type=jnp.float32)


def _tc_moments(domain2d, feats):
    return pl.pallas_call(
        _mm_body,
        grid=(_B // _BLK,),
        in_specs=[
            pl.BlockSpec((_BLK, 1), lambda i: (i, 0)),
            pl.BlockSpec((_BLK, _D), lambda i: (i, 0)),
        ],
        out_specs=pl.BlockSpec((_M, _D, _D), lambda i: (0, 0, 0)),
        out_shape=jax.ShapeDtypeStruct((_M, _D, _D), jnp.float32),
        compiler_params=pltpu.CompilerParams(dimension_semantics=("arbitrary",)),
    )(domain2d, feats)


def _outer(v):
    return lax.dot_general(v, v, (((0,), (0,)), ((), ())),
                           preferred_element_type=jnp.float32)


def _ep_body(sums_ref, cnts_ref, t_ref, out_ref):
    sums = jnp.concatenate([sums_ref[0], sums_ref[1]],
                           axis=-1).reshape(_M, _CP, _D)
    cnts = (cnts_ref[...].reshape(_NS, _NSEGP // 128, 128).sum(axis=0)
            .reshape(_M, _CP // 128, 128).reshape(_M, _CP))

    csafe = jnp.maximum(cnts, 1.0)
    mu = sums / csafe[:, :, None]
    presf = (cnts > 0.0).astype(jnp.float32)

    anchors = (1.0 - _MOM) * mu * presf[:, :, None]
    ag = jnp.zeros((_CP, _D), jnp.float32)
    for d in range(_M):
        upd = _MOM * ag + (1.0 - _MOM) * mu[d]
        pd = presf[d][:, None]
        ag = pd * upd + (1.0 - pd) * ag
    per = jnp.sum((anchors - ag[None]) ** 2, axis=-1) / _D
    nvalid = jnp.sum(presf)
    caa = jnp.where(nvalid > 0,
                    jnp.sum(per * presf) / jnp.maximum(nvalid, 1.0),
                    0.0)

    t = t_ref[...]
    s2 = [(t[0] + sg1 * t[1] + sg2 * t[2] + sg1 * sg2 * t[3]) * 0.25
          for d in range(_M)
          for sg1, sg2 in [(2 * (d & 1) - 1, 2 * (d >> 1) - 1)]]
    s2_tot = t[0]

    tot = jnp.sum(sums, axis=(0, 1)).reshape(1, _D)
    mu_g = tot / _B
    cov = (s2_tot - _B * _outer(mu_g)) / (_B + 1e-6)
    rows = lax.broadcasted_iota(jnp.int32, (_D, _D), 0)
    cols = lax.broadcasted_iota(jnp.int32, (_D, _D), 1)
    eye = (rows == cols).astype(jnp.float32)
    g_mean = (1.0 - _MOM) * mu_g
    g_cov = _MOM * eye + (1.0 - _MOM) * cov

    loss = jnp.float32(0.0)
    nval = jnp.float32(0.0)
    for d in range(_M):
        cnt = jnp.sum(cnts[d])
        s_row = jnp.sum(sums[d], axis=0).reshape(1, _D)
        mu_d = s_row / jnp.maximum(cnt, 1.0)
        cov_d = (s2[d] - cnt * _outer(mu_d)) / (cnt + 1e-6)
        l_d = jnp.mean((mu_d - g_mean) ** 2) + jnp.mean((cov_d - g_cov) ** 2)
        has = (cnt > 0).astype(jnp.float32)
        loss = loss + has * l_d
        nval = nval + has
    stats = jnp.where(nval > 0, loss / jnp.maximum(nval, 1.0), 0.0)

    out_ref[...] = jnp.full((1, 1), caa + stats, jnp.float32)


def _tc_epilogue(sums_p, cnts_p, s2):
    return pl.pallas_call(
        _ep_body,
        out_shape=jax.ShapeDtypeStruct((1, 1), jnp.float32),
    )(sums_p, cnts_p, s2)


def kernel(feats, labels, domain_ids):
    s2 = _tc_moments(domain_ids.astype(jnp.int8).reshape(_B, 1), feats)
    feats_t = (feats.reshape(_B // 8, 8, _H, 128)
               .transpose(0, 2, 1, 3).reshape(_B * _H, 128))
    sums_p, cnts_p = _get_sc_segsum()(feats_t, labels, domain_ids)
    loss = _tc_epilogue(sums_p,
                        cnts_p.reshape(_NS * _NSEGP // 128, 128),
                        s2)
    return loss.reshape(())


# speedup vs baseline: 1.9666x; 1.0228x over previous
import functools

import jax
import jax.numpy as jnp
from jax import lax
from jax.experimental import pallas as pl
from jax.experimental.pallas import tpu as pltpu
from jax.experimental.pallas import tpu_sc as plsc

_C = 1000
_D = 256
_M = 4
_MOM = 0.9
_B = 16384
_CP = 1024
_NSEGP = _M * _CP
_H = _D // 128

_NC = 2
_NS = 16
_L = 16
_NW = _NC * _NS
_RPT2 = _B // _NS
_CHUNK2 = 128
_NCHUNK2 = _RPT2 // _CHUNK2


def _sc_body(feats_hbm, labels_hbm, domains_hbm, out_sums, out_cnts,
             buf0, buf1, gidx, seg2d, lab_v, dom_v, cnt_v, acc_sh, sem0, sem1):
    cid = lax.axis_index("c")
    sid = lax.axis_index("s")
    rbase = sid * _RPT2

    iota = lax.broadcasted_iota(jnp.int32, (_L,), 0)
    for k in range(_NCHUNK2):
        for i in range(_CHUNK2 // _L):
            rv = rbase + k * _CHUNK2 + i * _L + iota
            gidx[k, pl.ds(i * _L, _L)] = ((rv >> 3) << 4) + 8 * cid + (rv & 7)

    pltpu.sync_copy(labels_hbm.at[pl.ds(rbase, _RPT2)], lab_v)
    pltpu.sync_copy(domains_hbm.at[pl.ds(rbase, _RPT2)], dom_v)

    zrow = jnp.zeros((_L,), jnp.float32)

    def _fill_row(i, carry):
        for k in range(128 // _L):
            buf0[i, pl.ds(k * _L, _L)] = zrow
        cnt_v[pl.ds(i * _L, _L)] = zrow
        cnt_v[pl.ds((i + 128) * _L, _L)] = zrow
        return carry

    lax.fori_loop(0, _CHUNK2, _fill_row, 0)

    r0 = sid * (_NSEGP // _NS)
    pltpu.sync_copy(buf0, acc_sh.at[pl.ds(r0, _CHUNK2)])
    pltpu.sync_copy(buf0, acc_sh.at[pl.ds(r0 + _CHUNK2, _CHUNK2)])

    onesv = zrow + 1.0
    for i in range(_RPT2 // _L):
        seg = dom_v[pl.ds(i * _L, _L)] * _CP + lab_v[pl.ds(i * _L, _L)]
        seg2d[i // 8, pl.ds((i % 8) * _L, _L)] = seg
        plsc.addupdate_scatter(cnt_v, [seg], onesv)

    cp0 = pltpu.async_copy(feats_hbm.at[gidx.at[0]], buf1, sem1)

    plsc.subcore_barrier()

    descs = [cp0]
    for k in range(_NCHUNK2):
        descs[k].wait()
        if k + 1 < _NCHUNK2:
            nref = buf0 if (k + 1) % 2 == 1 else buf1
            sem = sem0 if (k + 1) % 2 == 1 else sem1
            descs.append(pltpu.async_copy(
                feats_hbm.at[gidx.at[k + 1]], nref, sem))
        cref = buf1 if k % 2 == 0 else buf0
        pltpu.sync_copy(cref, acc_sh.at[seg2d.at[k]], add=True)

    plsc.subcore_barrier()

    pltpu.sync_copy(acc_sh.at[pl.ds(r0, _NSEGP // _NS)],
                    out_sums.at[cid, pl.ds(r0, _NSEGP // _NS)])

    @pl.when(cid == 0)
    def _():
        pltpu.sync_copy(cnt_v, out_cnts.at[sid])


@functools.cache
def _get_sc_segsum():
    return pl.kernel(
        _sc_body,
        out_type=(
            jax.ShapeDtypeStruct((_H, _NSEGP, 128), jnp.float32),
            jax.ShapeDtypeStruct((_NS, _NSEGP), jnp.float32),
        ),
        mesh=plsc.VectorSubcoreMesh(core_axis_name="c", subcore_axis_name="s"),
        compiler_params=pltpu.CompilerParams(use_tc_tiling_on_sc=False,
                                             needs_layout_passes=False),
        scratch_types=[
            pltpu.VMEM((_CHUNK2, 128), jnp.float32),
            pltpu.VMEM((_CHUNK2, 128), jnp.float32),
            pltpu.VMEM((_NCHUNK2, 128), jnp.int32),
            pltpu.VMEM((_NCHUNK2, 128), jnp.int32),
            pltpu.VMEM((_RPT2,), jnp.int32),
            pltpu.VMEM((_RPT2,), jnp.int32),
            pltpu.VMEM((_NSEGP,), jnp.float32),
            pltpu.VMEM_SHARED((_NSEGP, 128), jnp.float32),
            pltpu.SemaphoreType.DMA,
            pltpu.SemaphoreType.DMA,
        ],
    )


_BLK = 4096


def _mm_body(dom_ref, x_ref, t_ref):
    i = pl.program_id(0)

    @pl.when(i == 0)
    def _init():
        t_ref[...] = jnp.zeros_like(t_ref)

    x = x_ref[...].astype(jnp.bfloat16)
    dom = dom_ref[...].astype(jnp.int32)
    xs1 = jnp.where((dom & 1) == 1, x, -x)
    xs2 = jnp.where((dom & 2) == 2, x, -x)
    dn = (((0,), (0,)), ((), ()))
    t_ref[0] += lax.dot_general(x, x, dn, preferred_element_type=jnp.float32)
    t_ref[1] += lax.dot_general(x, xs1, dn, preferred_element_type=jnp.float32)
    t_ref[2] += lax.dot_general(x, xs2, dn, preferred_element_type=jnp.float32)
    t_ref[3] += lax.dot_general(xs1, xs2, dn, preferred_element_type=jnp.float32)


def _tc_moments(domain2d, feats):
    return pl.pallas_call(
        _mm_body,
        grid=(_B // _BLK,),
        in_specs=[
            pl.BlockSpec((_BLK, 1), lambda i: (i, 0)),
            pl.BlockSpec((_BLK, _D), lambda i: (i, 0)),
        ],
        out_specs=pl.BlockSpec((_M, _D, _D), lambda i: (0, 0, 0)),
        out_shape=jax.ShapeDtypeStruct((_M, _D, _D), jnp.float32),
        compiler_params=pltpu.CompilerParams(dimension_semantics=("arbitrary",)),
    )(domain2d, feats)


def _outer(v):
    return lax.dot_general(v, v, (((0,), (0,)), ((), ())),
                           preferred_element_type=jnp.float32)


def _ep_body(sums_ref, cnts_ref, t_ref, out_ref):
    sums = jnp.concatenate([sums_ref[0], sums_ref[1]],
                           axis=-1).reshape(_M, _CP, _D)
    cnts = (cnts_ref[...].reshape(_NS, _NSEGP // 128, 128).sum(axis=0)
            .reshape(_M, _CP // 128, 128).reshape(_M, _CP))

    csafe = jnp.maximum(cnts, 1.0)
    mu = sums / csafe[:, :, None]
    presf = (cnts > 0.0).astype(jnp.float32)

    anchors = (1.0 - _MOM) * mu * presf[:, :, None]
    ag = jnp.zeros((_CP, _D), jnp.float32)
    for d in range(_M):
        upd = _MOM * ag + (1.0 - _MOM) * mu[d]
        pd = presf[d][:, None]
        ag = pd * upd + (1.0 - pd) * ag
    per = jnp.sum((anchors - ag[None]) ** 2, axis=-1) / _D
    nvalid = jnp.sum(presf)
    caa = jnp.where(nvalid > 0,
                    jnp.sum(per * presf) / jnp.maximum(nvalid, 1.0),
                    0.0)

    t = t_ref[...]
    s2 = [(t[0] + sg1 * t[1] + sg2 * t[2] + sg1 * sg2 * t[3]) * 0.25
          for d in range(_M)
          for sg1, sg2 in [(2 * (d & 1) - 1, 2 * (d >> 1) - 1)]]
    s2_tot = t[0]

    tot = jnp.sum(sums, axis=(0, 1)).reshape(1, _D)
    mu_g = tot / _B
    cov = (s2_tot - _B * _outer(mu_g)) / (_B + 1e-6)
    rows = lax.broadcasted_iota(jnp.int32, (_D, _D), 0)
    cols = lax.broadcasted_iota(jnp.int32, (_D, _D), 1)
    eye = (rows == cols).astype(jnp.float32)
    g_mean = (1.0 - _MOM) * mu_g
    g_cov = _MOM * eye + (1.0 - _MOM) * cov

    loss = jnp.float32(0.0)
    nval = jnp.float32(0.0)
    for d in range(_M):
        cnt = jnp.sum(cnts[d])
        s_row = jnp.sum(sums[d], axis=0).reshape(1, _D)
        mu_d = s_row / jnp.maximum(cnt, 1.0)
        cov_d = (s2[d] - cnt * _outer(mu_d)) / (cnt + 1e-6)
        l_d = jnp.mean((mu_d - g_mean) ** 2) + jnp.mean((cov_d - g_cov) ** 2)
        has = (cnt > 0).astype(jnp.float32)
        loss = loss + has * l_d
        nval = nval + has
    stats = jnp.where(nval > 0, loss / jnp.maximum(nval, 1.0), 0.0)

    out_ref[...] = jnp.full((1, 1), caa + stats, jnp.float32)


def _tc_epilogue(sums_p, cnts_p, s2):
    return pl.pallas_call(
        _ep_body,
        out_shape=jax.ShapeDtypeStruct((1, 1), jnp.float32),
    )(sums_p, cnts_p, s2)


def kernel(feats, labels, domain_ids):
    s2 = _tc_moments(domain_ids.astype(jnp.int8).reshape(_B, 1), feats)
    feats_t = (feats.reshape(_B // 8, 8, _H, 128)
               .transpose(0, 2, 1, 3).reshape(_B * _H, 128))
    sums_p, cnts_p = _get_sc_segsum()(feats_t, labels, domain_ids)
    loss = _tc_epilogue(sums_p,
                        cnts_p.reshape(_NS * _NSEGP // 128, 128),
                        s2)
    return loss.reshape(())


# speedup vs baseline: 1.9918x; 1.0128x over previous
import functools

import jax
import jax.numpy as jnp
from jax import lax
from jax.experimental import pallas as pl
from jax.experimental.pallas import tpu as pltpu
from jax.experimental.pallas import tpu_sc as plsc

_C = 1000
_D = 256
_M = 4
_MOM = 0.9
_B = 16384
_CP = 1024
_NSEGP = _M * _CP
_H = _D // 128

_NC = 2
_NS = 16
_L = 16
_NW = _NC * _NS
_RPT2 = _B // _NS
_CHUNK2 = 128
_NCHUNK2 = _RPT2 // _CHUNK2


def _sc_body(feats_hbm, labels_hbm, domains_hbm, out_sums, out_cnts,
             buf0, buf1, gidx, seg2d, lab_v, dom_v, cnt_v, acc_sh, sem0, sem1):
    cid = lax.axis_index("c")
    sid = lax.axis_index("s")
    rbase = sid * _RPT2

    iota = lax.broadcasted_iota(jnp.int32, (_L,), 0)

    def _gidx_row(q, carry):
        rv = rbase + q * _L + iota
        gidx[q // (_CHUNK2 // _L), pl.ds((q % (_CHUNK2 // _L)) * _L, _L)] = (
            ((rv >> 3) << 4) + 8 * cid + (rv & 7))
        return carry

    lax.fori_loop(0, _RPT2 // _L, _gidx_row, 0)

    pltpu.sync_copy(labels_hbm.at[pl.ds(rbase, _RPT2)], lab_v)
    pltpu.sync_copy(domains_hbm.at[pl.ds(rbase, _RPT2)], dom_v)

    zrow = jnp.zeros((_L,), jnp.float32)

    def _fill_row(i, carry):
        for k in range(128 // _L):
            buf0[i, pl.ds(k * _L, _L)] = zrow
        cnt_v[pl.ds(i * _L, _L)] = zrow
        cnt_v[pl.ds((i + 128) * _L, _L)] = zrow
        return carry

    lax.fori_loop(0, _CHUNK2, _fill_row, 0)

    r0 = sid * (_NSEGP // _NS)
    pltpu.sync_copy(buf0, acc_sh.at[pl.ds(r0, _CHUNK2)])
    pltpu.sync_copy(buf0, acc_sh.at[pl.ds(r0 + _CHUNK2, _CHUNK2)])

    onesv = zrow + 1.0
    half_groups = _RPT2 // _L // 2
    cbase = cid * half_groups

    def _seg_row(i, carry):
        seg = dom_v[pl.ds(i * _L, _L)] * _CP + lab_v[pl.ds(i * _L, _L)]
        seg2d[i // 8, pl.ds((i % 8) * _L, _L)] = seg
        return carry

    lax.fori_loop(0, _RPT2 // _L, _seg_row, 0)

    def _cnt_row(i, carry):
        seg = (dom_v[pl.ds((cbase + i) * _L, _L)] * _CP
               + lab_v[pl.ds((cbase + i) * _L, _L)])
        plsc.addupdate_scatter(cnt_v, [seg], onesv)
        return carry

    lax.fori_loop(0, half_groups, _cnt_row, 0)

    cp0 = pltpu.async_copy(feats_hbm.at[gidx.at[0]], buf1, sem1)

    plsc.subcore_barrier()

    descs = [cp0]
    for k in range(_NCHUNK2):
        descs[k].wait()
        if k + 1 < _NCHUNK2:
            nref = buf0 if (k + 1) % 2 == 1 else buf1
            sem = sem0 if (k + 1) % 2 == 1 else sem1
            descs.append(pltpu.async_copy(
                feats_hbm.at[gidx.at[k + 1]], nref, sem))
        cref = buf1 if k % 2 == 0 else buf0
        pltpu.sync_copy(cref, acc_sh.at[seg2d.at[k]], add=True)

    plsc.subcore_barrier()

    pltpu.sync_copy(acc_sh.at[pl.ds(r0, _NSEGP // _NS)],
                    out_sums.at[cid, pl.ds(r0, _NSEGP // _NS)])
    pltpu.sync_copy(cnt_v, out_cnts.at[sid * _NC + cid])


@functools.cache
def _get_sc_segsum():
    return pl.kernel(
        _sc_body,
        out_type=(
            jax.ShapeDtypeStruct((_H, _NSEGP, 128), jnp.float32),
            jax.ShapeDtypeStruct((_NW, _NSEGP), jnp.float32),
        ),
        mesh=plsc.VectorSubcoreMesh(core_axis_name="c", subcore_axis_name="s"),
        compiler_params=pltpu.CompilerParams(use_tc_tiling_on_sc=False,
                                             needs_layout_passes=False),
        scratch_types=[
            pltpu.VMEM((_CHUNK2, 128), jnp.float32),
            pltpu.VMEM((_CHUNK2, 128), jnp.float32),
            pltpu.VMEM((_NCHUNK2, 128), jnp.int32),
            pltpu.VMEM((_NCHUNK2, 128), jnp.int32),
            pltpu.VMEM((_RPT2,), jnp.int32),
            pltpu.VMEM((_RPT2,), jnp.int32),
            pltpu.VMEM((_NSEGP,), jnp.float32),
            pltpu.VMEM_SHARED((_NSEGP, 128), jnp.float32),
            pltpu.SemaphoreType.DMA,
            pltpu.SemaphoreType.DMA,
        ],
    )


_BLK = 4096


def _mm_body(dom_ref, x_ref, t_ref):
    i = pl.program_id(0)

    @pl.when(i == 0)
    def _init():
        t_ref[...] = jnp.zeros_like(t_ref)

    x = x_ref[...].astype(jnp.bfloat16)
    dom = dom_ref[...].astype(jnp.int32)
    xs1 = jnp.where((dom & 1) == 1, x, -x)
    xs2 = jnp.where((dom & 2) == 2, x, -x)
    dn = (((0,), (0,)), ((), ()))
    t_ref[0] += lax.dot_general(x, x, dn, preferred_element_type=jnp.float32)
    t_ref[1] += lax.dot_general(x, xs1, dn, preferred_element_type=jnp.float32)
    t_ref[2] += lax.dot_general(x, xs2, dn, preferred_element_type=jnp.float32)
    t_ref[3] += lax.dot_general(xs1, xs2, dn, preferred_element_type=jnp.float32)


def _tc_moments(domain2d, feats):
    return pl.pallas_call(
        _mm_body,
        grid=(_B // _BLK,),
        in_specs=[
            pl.BlockSpec((_BLK, 1), lambda i: (i, 0)),
            pl.BlockSpec((_BLK, _D), lambda i: (i, 0)),
        ],
        out_specs=pl.BlockSpec((_M, _D, _D), lambda i: (0, 0, 0)),
        out_shape=jax.ShapeDtypeStruct((_M, _D, _D), jnp.float32),
        compiler_params=pltpu.CompilerParams(dimension_semantics=("arbitrary",)),
    )(domain2d, feats)


def _outer(v):
    return lax.dot_general(v, v, (((0,), (0,)), ((), ())),
                           preferred_element_type=jnp.float32)


def _ep_body(sums_ref, cnts_ref, t_ref, out_ref):
    sums = jnp.concatenate([sums_ref[0], sums_ref[1]],
                           axis=-1).reshape(_M, _CP, _D)
    cnts = (cnts_ref[...].reshape(_NW, _NSEGP // 128, 128).sum(axis=0)
            .reshape(_M, _CP // 128, 128).reshape(_M, _CP))

    csafe = jnp.maximum(cnts, 1.0)
    mu = sums / csafe[:, :, None]
    presf = (cnts > 0.0).astype(jnp.float32)

    anchors = (1.0 - _MOM) * mu * presf[:, :, None]
    ag = jnp.zeros((_CP, _D), jnp.float32)
    for d in range(_M):
        upd = _MOM * ag + (1.0 - _MOM) * mu[d]
        pd = presf[d][:, None]
        ag = pd * upd + (1.0 - pd) * ag
    per = jnp.sum((anchors - ag[None]) ** 2, axis=-1) / _D
    nvalid = jnp.sum(presf)
    caa = jnp.where(nvalid > 0,
                    jnp.sum(per * presf) / jnp.maximum(nvalid, 1.0),
                    0.0)

    t = t_ref[...]
    s2 = [(t[0] + sg1 * t[1] + sg2 * t[2] + sg1 * sg2 * t[3]) * 0.25
          for d in range(_M)
          for sg1, sg2 in [(2 * (d & 1) - 1, 2 * (d >> 1) - 1)]]
    s2_tot = t[0]

    tot = jnp.sum(sums, axis=(0, 1)).reshape(1, _D)
    mu_g = tot / _B
    cov = (s2_tot - _B * _outer(mu_g)) / (_B + 1e-6)
    rows = lax.broadcasted_iota(jnp.int32, (_D, _D), 0)
    cols = lax.broadcasted_iota(jnp.int32, (_D, _D), 1)
    eye = (rows == cols).astype(jnp.float32)
    g_mean = (1.0 - _MOM) * mu_g
    g_cov = _MOM * eye + (1.0 - _MOM) * cov

    loss = jnp.float32(0.0)
    nval = jnp.float32(0.0)
    for d in range(_M):
        cnt = jnp.sum(cnts[d])
        s_row = jnp.sum(sums[d], axis=0).reshape(1, _D)
        mu_d = s_row / jnp.maximum(cnt, 1.0)
        cov_d = (s2[d] - cnt * _outer(mu_d)) / (cnt + 1e-6)
        l_d = jnp.mean((mu_d - g_mean) ** 2) + jnp.mean((cov_d - g_cov) ** 2)
        has = (cnt > 0).astype(jnp.float32)
        loss = loss + has * l_d
        nval = nval + has
    stats = jnp.where(nval > 0, loss / jnp.maximum(nval, 1.0), 0.0)

    out_ref[...] = jnp.full((1, 1), caa + stats, jnp.float32)


def _tc_epilogue(sums_p, cnts_p, s2):
    return pl.pallas_call(
        _ep_body,
        out_shape=jax.ShapeDtypeStruct((1, 1), jnp.float32),
    )(sums_p, cnts_p, s2)


def kernel(feats, labels, domain_ids):
    s2 = _tc_moments(domain_ids.astype(jnp.int8).reshape(_B, 1), feats)
    feats_t = (feats.reshape(_B // 8, 8, _H, 128)
               .transpose(0, 2, 1, 3).reshape(_B * _H, 128))
    sums_p, cnts_p = _get_sc_segsum()(feats_t, labels, domain_ids)
    loss = _tc_epilogue(sums_p,
                        cnts_p.reshape(_NW * _NSEGP // 128, 128),
                        s2)
    return loss.reshape(())


# speedup vs baseline: 2.0432x; 1.0258x over previous
import functools

import jax
import jax.numpy as jnp
from jax import lax
from jax.experimental import pallas as pl
from jax.experimental.pallas import tpu as pltpu
from jax.experimental.pallas import tpu_sc as plsc

_C = 1000
_D = 256
_M = 4
_MOM = 0.9
_B = 16384
_CP = 1024
_NSEGP = _M * _CP
_H = _D // 128

_NC = 2
_NS = 16
_L = 16
_NW = _NC * _NS
_RPT2 = _B // _NS
_CHUNK2 = 128
_NCHUNK2 = _RPT2 // _CHUNK2


def _sc_body(feats_hbm, labels_hbm, domains_hbm, out_sums, out_cnts,
             zbuf, buf0, buf1, buf2, buf3, gidx, seg2d, lab_v, dom_v, cnt_v,
             acc_sh, gs0, gs1, gs2, gs3, ss0, ss1, ss2, ss3):
    cid = lax.axis_index("c")
    sid = lax.axis_index("s")
    rbase = sid * _RPT2

    iota = lax.broadcasted_iota(jnp.int32, (_L,), 0)

    def _gidx_row(q, carry):
        rv = rbase + q * _L + iota
        gidx[q // (_CHUNK2 // _L), pl.ds((q % (_CHUNK2 // _L)) * _L, _L)] = (
            ((rv >> 3) << 4) + 8 * cid + (rv & 7))
        return carry

    lax.fori_loop(0, _RPT2 // _L, _gidx_row, 0)

    pltpu.sync_copy(labels_hbm.at[pl.ds(rbase, _RPT2)], lab_v)
    pltpu.sync_copy(domains_hbm.at[pl.ds(rbase, _RPT2)], dom_v)

    zrow = jnp.zeros((_L,), jnp.float32)

    def _fill_row(i, carry):
        for k in range(128 // _L):
            zbuf[i, pl.ds(k * _L, _L)] = zrow
        cnt_v[pl.ds(i * _L, _L)] = zrow
        cnt_v[pl.ds((i + 128) * _L, _L)] = zrow
        return carry

    lax.fori_loop(0, _CHUNK2, _fill_row, 0)

    r0 = sid * (_NSEGP // _NS)
    pltpu.sync_copy(zbuf, acc_sh.at[pl.ds(r0, _CHUNK2)])
    pltpu.sync_copy(zbuf, acc_sh.at[pl.ds(r0 + _CHUNK2, _CHUNK2)])

    onesv = zrow + 1.0
    half_groups = _RPT2 // _L // 2
    cbase = cid * half_groups

    def _seg_row(i, carry):
        seg = dom_v[pl.ds(i * _L, _L)] * _CP + lab_v[pl.ds(i * _L, _L)]
        seg2d[i // 8, pl.ds((i % 8) * _L, _L)] = seg
        return carry

    lax.fori_loop(0, _RPT2 // _L, _seg_row, 0)

    def _cnt_row(i, carry):
        seg = (dom_v[pl.ds((cbase + i) * _L, _L)] * _CP
               + lab_v[pl.ds((cbase + i) * _L, _L)])
        plsc.addupdate_scatter(cnt_v, [seg], onesv)
        return carry

    lax.fori_loop(0, half_groups, _cnt_row, 0)

    bufs = [buf0, buf1, buf2, buf3]
    gsems = [gs0, gs1, gs2, gs3]
    ssems = [ss0, ss1, ss2, ss3]
    gd = [pltpu.async_copy(feats_hbm.at[gidx.at[k]], bufs[k], gsems[k])
          for k in range(4)]
    gd += [None] * (_NCHUNK2 - 4)
    sd = [None] * _NCHUNK2

    plsc.subcore_barrier()

    for k in range(_NCHUNK2):
        nk = k + 1
        if nk < _NCHUNK2 and nk >= 4:
            sd[nk - 4].wait()
            gd[nk] = pltpu.async_copy(
                feats_hbm.at[gidx.at[nk]], bufs[nk % 4], gsems[nk % 4])
        gd[k].wait()
        sd[k] = pltpu.async_copy(bufs[k % 4], acc_sh.at[seg2d.at[k]],
                                 ssems[k % 4], add=True)
    for k in range(_NCHUNK2 - 3, _NCHUNK2):
        sd[k].wait()

    plsc.subcore_barrier()

    pltpu.sync_copy(acc_sh.at[pl.ds(r0, _NSEGP // _NS)],
                    out_sums.at[cid, pl.ds(r0, _NSEGP // _NS)])
    pltpu.sync_copy(cnt_v, out_cnts.at[sid * _NC + cid])


@functools.cache
def _get_sc_segsum():
    return pl.kernel(
        _sc_body,
        out_type=(
            jax.ShapeDtypeStruct((_H, _NSEGP, 128), jnp.float32),
            jax.ShapeDtypeStruct((_NW, _NSEGP), jnp.float32),
        ),
        mesh=plsc.VectorSubcoreMesh(core_axis_name="c", subcore_axis_name="s"),
        compiler_params=pltpu.CompilerParams(use_tc_tiling_on_sc=False,
                                             needs_layout_passes=False),
        scratch_types=[
            pltpu.VMEM((_CHUNK2, 128), jnp.float32),
            pltpu.VMEM((_CHUNK2, 128), jnp.float32),
            pltpu.VMEM((_CHUNK2, 128), jnp.float32),
            pltpu.VMEM((_CHUNK2, 128), jnp.float32),
            pltpu.VMEM((_CHUNK2, 128), jnp.float32),
            pltpu.VMEM((_NCHUNK2, 128), jnp.int32),
            pltpu.VMEM((_NCHUNK2, 128), jnp.int32),
            pltpu.VMEM((_RPT2,), jnp.int32),
            pltpu.VMEM((_RPT2,), jnp.int32),
            pltpu.VMEM((_NSEGP,), jnp.float32),
            pltpu.VMEM_SHARED((_NSEGP, 128), jnp.float32),
            pltpu.SemaphoreType.DMA,
            pltpu.SemaphoreType.DMA,
            pltpu.SemaphoreType.DMA,
            pltpu.SemaphoreType.DMA,
            pltpu.SemaphoreType.DMA,
            pltpu.SemaphoreType.DMA,
            pltpu.SemaphoreType.DMA,
            pltpu.SemaphoreType.DMA,
        ],
    )


_BLK = 4096


def _mm_body(dom_ref, x_ref, t_ref):
    i = pl.program_id(0)

    @pl.when(i == 0)
    def _init():
        t_ref[...] = jnp.zeros_like(t_ref)

    x = x_ref[...].astype(jnp.bfloat16)
    dom = dom_ref[...].astype(jnp.int32)
    xs1 = jnp.where((dom & 1) == 1, x, -x)
    xs2 = jnp.where((dom & 2) == 2, x, -x)
    dn = (((0,), (0,)), ((), ()))
    t_ref[0] += lax.dot_general(x, x, dn, preferred_element_type=jnp.float32)
    t_ref[1] += lax.dot_general(x, xs1, dn, preferred_element_type=jnp.float32)
    t_ref[2] += lax.dot_general(x, xs2, dn, preferred_element_type=jnp.float32)
    t_ref[3] += lax.dot_general(xs1, xs2, dn, preferred_element_type=jnp.float32)


def _tc_moments(domain2d, feats):
    return pl.pallas_call(
        _mm_body,
        grid=(_B // _BLK,),
        in_specs=[
            pl.BlockSpec((_BLK, 1), lambda i: (i, 0)),
            pl.BlockSpec((_BLK, _D), lambda i: (i, 0)),
        ],
        out_specs=pl.BlockSpec((_M, _D, _D), lambda i: (0, 0, 0)),
        out_shape=jax.ShapeDtypeStruct((_M, _D, _D), jnp.float32),
        compiler_params=pltpu.CompilerParams(dimension_semantics=("arbitrary",)),
    )(domain2d, feats)


def _outer(v):
    return lax.dot_general(v, v, (((0,), (0,)), ((), ())),
                           preferred_element_type=jnp.float32)


def _ep_body(sums_ref, cnts_ref, t_ref, out_ref):
    sums = jnp.concatenate([sums_ref[0], sums_ref[1]],
                           axis=-1).reshape(_M, _CP, _D)
    cnts = (cnts_ref[...].reshape(_NW, _NSEGP // 128, 128).sum(axis=0)
            .reshape(_M, _CP // 128, 128).reshape(_M, _CP))

    csafe = jnp.maximum(cnts, 1.0)
    mu = sums / csafe[:, :, None]
    presf = (cnts > 0.0).astype(jnp.float32)

    anchors = (1.0 - _MOM) * mu * presf[:, :, None]
    ag = jnp.zeros((_CP, _D), jnp.float32)
    for d in range(_M):
        upd = _MOM * ag + (1.0 - _MOM) * mu[d]
        pd = presf[d][:, None]
        ag = pd * upd + (1.0 - pd) * ag
    per = jnp.sum((anchors - ag[None]) ** 2, axis=-1) / _D
    nvalid = jnp.sum(presf)
    caa = jnp.where(nvalid > 0,
                    jnp.sum(per * presf) / jnp.maximum(nvalid, 1.0),
                    0.0)

    t = t_ref[...]
    s2 = [(t[0] + sg1 * t[1] + sg2 * t[2] + sg1 * sg2 * t[3]) * 0.25
          for d in range(_M)
          for sg1, sg2 in [(2 * (d & 1) - 1, 2 * (d >> 1) - 1)]]
    s2_tot = t[0]

    tot = jnp.sum(sums, axis=(0, 1)).reshape(1, _D)
    mu_g = tot / _B
    cov = (s2_tot - _B * _outer(mu_g)) / (_B + 1e-6)
    rows = lax.broadcasted_iota(jnp.int32, (_D, _D), 0)
    cols = lax.broadcasted_iota(jnp.int32, (_D, _D), 1)
    eye = (rows == cols).astype(jnp.float32)
    g_mean = (1.0 - _MOM) * mu_g
    g_cov = _MOM * eye + (1.0 - _MOM) * cov

    loss = jnp.float32(0.0)
    nval = jnp.float32(0.0)
    for d in range(_M):
        cnt = jnp.sum(cnts[d])
        s_row = jnp.sum(sums[d], axis=0).reshape(1, _D)
        mu_d = s_row / jnp.maximum(cnt, 1.0)
        cov_d = (s2[d] - cnt * _outer(mu_d)) / (cnt + 1e-6)
        l_d = jnp.mean((mu_d - g_mean) ** 2) + jnp.mean((cov_d - g_cov) ** 2)
        has = (cnt > 0).astype(jnp.float32)
        loss = loss + has * l_d
        nval = nval + has
    stats = jnp.where(nval > 0, loss / jnp.maximum(nval, 1.0), 0.0)

    out_ref[...] = jnp.full((1, 1), caa + stats, jnp.float32)


def _tc_epilogue(sums_p, cnts_p, s2):
    return pl.pallas_call(
        _ep_body,
        out_shape=jax.ShapeDtypeStruct((1, 1), jnp.float32),
    )(sums_p, cnts_p, s2)


def kernel(feats, labels, domain_ids):
    s2 = _tc_moments(domain_ids.astype(jnp.int8).reshape(_B, 1), feats)
    feats_t = (feats.reshape(_B // 8, 8, _H, 128)
               .transpose(0, 2, 1, 3).reshape(_B * _H, 128))
    sums_p, cnts_p = _get_sc_segsum()(feats_t, labels, domain_ids)
    loss = _tc_epilogue(sums_p,
                        cnts_p.reshape(_NW * _NSEGP // 128, 128),
                        s2)
    return loss.reshape(())


# speedup vs baseline: 2.0707x; 1.0134x over previous
import functools

import jax
import jax.numpy as jnp
from jax import lax
from jax.experimental import pallas as pl
from jax.experimental.pallas import tpu as pltpu
from jax.experimental.pallas import tpu_sc as plsc

_C = 1000
_D = 256
_M = 4
_MOM = 0.9
_B = 16384
_CP = 1024
_NSEGP = _M * _CP
_H = _D // 128

_NC = 2
_NS = 16
_L = 16
_NW = _NC * _NS
_RPT2 = _B // _NS
_CHUNK2 = 128
_NCHUNK2 = _RPT2 // _CHUNK2


def _sc_body(feats_hbm, labels_hbm, domains_hbm, out_sums, out_cnts,
             zbuf, buf0, buf1, buf2, buf3, gidx, seg2d, lab_v, dom_v, cnt_v,
             acc_sh, gs0, gs1, gs2, gs3, ss0, ss1, ss2, ss3):
    cid = lax.axis_index("c")
    sid = lax.axis_index("s")
    rbase = sid * _RPT2

    iota = lax.broadcasted_iota(jnp.int32, (_L,), 0)

    def _gidx_row(q, carry):
        rv = rbase + q * _L + iota
        gidx[q // (_CHUNK2 // _L), pl.ds((q % (_CHUNK2 // _L)) * _L, _L)] = (
            ((rv >> 3) << 4) + 8 * cid + (rv & 7))
        return carry

    lax.fori_loop(0, _RPT2 // _L, _gidx_row, 0)

    pltpu.sync_copy(labels_hbm.at[pl.ds(rbase, _RPT2)], lab_v)
    pltpu.sync_copy(domains_hbm.at[pl.ds(rbase, _RPT2)], dom_v)

    zrow = jnp.zeros((_L,), jnp.float32)

    def _fill_row(i, carry):
        for k in range(128 // _L):
            zbuf[i, pl.ds(k * _L, _L)] = zrow
        cnt_v[pl.ds(i * _L, _L)] = zrow
        cnt_v[pl.ds((i + 128) * _L, _L)] = zrow
        return carry

    lax.fori_loop(0, _CHUNK2, _fill_row, 0)

    bufs = [buf0, buf1, buf2, buf3]
    gsems = [gs0, gs1, gs2, gs3]
    ssems = [ss0, ss1, ss2, ss3]
    gd = [pltpu.async_copy(feats_hbm.at[gidx.at[k]], bufs[k], gsems[k])
          for k in range(4)]
    gd += [None] * (_NCHUNK2 - 4)
    sd = [None] * _NCHUNK2

    r0 = sid * (_NSEGP // _NS)
    pltpu.sync_copy(zbuf, acc_sh.at[pl.ds(r0, _CHUNK2)])
    pltpu.sync_copy(zbuf, acc_sh.at[pl.ds(r0 + _CHUNK2, _CHUNK2)])

    onesv = zrow + 1.0
    half_groups = _RPT2 // _L // 2
    cbase = cid * half_groups

    def _seg_row(i, carry):
        seg = dom_v[pl.ds(i * _L, _L)] * _CP + lab_v[pl.ds(i * _L, _L)]
        seg2d[i // 8, pl.ds((i % 8) * _L, _L)] = seg
        return carry

    lax.fori_loop(0, _RPT2 // _L, _seg_row, 0)

    def _cnt_row(i, carry):
        seg = (dom_v[pl.ds((cbase + i) * _L, _L)] * _CP
               + lab_v[pl.ds((cbase + i) * _L, _L)])
        plsc.addupdate_scatter(cnt_v, [seg], onesv)
        return carry

    lax.fori_loop(0, half_groups, _cnt_row, 0)

    plsc.subcore_barrier()

    for k in range(_NCHUNK2):
        nk = k + 1
        if nk < _NCHUNK2 and nk >= 4:
            sd[nk - 4].wait()
            gd[nk] = pltpu.async_copy(
                feats_hbm.at[gidx.at[nk]], bufs[nk % 4], gsems[nk % 4])
        gd[k].wait()
        sd[k] = pltpu.async_copy(bufs[k % 4], acc_sh.at[seg2d.at[k]],
                                 ssems[k % 4], add=True)
    for k in range(_NCHUNK2 - 3, _NCHUNK2):
        sd[k].wait()

    plsc.subcore_barrier()

    pltpu.sync_copy(acc_sh.at[pl.ds(r0, _NSEGP // _NS)],
                    out_sums.at[cid, pl.ds(r0, _NSEGP // _NS)])
    pltpu.sync_copy(cnt_v, out_cnts.at[sid * _NC + cid])


@functools.cache
def _get_sc_segsum():
    return pl.kernel(
        _sc_body,
        out_type=(
            jax.ShapeDtypeStruct((_H, _NSEGP, 128), jnp.float32),
            jax.ShapeDtypeStruct((_NW, _NSEGP), jnp.float32),
        ),
        mesh=plsc.VectorSubcoreMesh(core_axis_name="c", subcore_axis_name="s"),
        compiler_params=pltpu.CompilerParams(use_tc_tiling_on_sc=False,
                                             needs_layout_passes=False),
        scratch_types=[
            pltpu.VMEM((_CHUNK2, 128), jnp.float32),
            pltpu.VMEM((_CHUNK2, 128), jnp.float32),
            pltpu.VMEM((_CHUNK2, 128), jnp.float32),
            pltpu.VMEM((_CHUNK2, 128), jnp.float32),
            pltpu.VMEM((_CHUNK2, 128), jnp.float32),
            pltpu.VMEM((_NCHUNK2, 128), jnp.int32),
            pltpu.VMEM((_NCHUNK2, 128), jnp.int32),
            pltpu.VMEM((_RPT2,), jnp.int32),
            pltpu.VMEM((_RPT2,), jnp.int32),
            pltpu.VMEM((_NSEGP,), jnp.float32),
            pltpu.VMEM_SHARED((_NSEGP, 128), jnp.float32),
            pltpu.SemaphoreType.DMA,
            pltpu.SemaphoreType.DMA,
            pltpu.SemaphoreType.DMA,
            pltpu.SemaphoreType.DMA,
            pltpu.SemaphoreType.DMA,
            pltpu.SemaphoreType.DMA,
            pltpu.SemaphoreType.DMA,
            pltpu.SemaphoreType.DMA,
        ],
    )


_BLK = 4096


def _mm_body(dom_ref, x_ref, t_ref):
    i = pl.program_id(0)

    @pl.when(i == 0)
    def _init():
        t_ref[...] = jnp.zeros_like(t_ref)

    x = x_ref[...].astype(jnp.bfloat16)
    dom = dom_ref[...].astype(jnp.int32)
    xs1 = jnp.where((dom & 1) == 1, x, -x)
    xs2 = jnp.where((dom & 2) == 2, x, -x)
    dn = (((0,), (0,)), ((), ()))
    t_ref[0] += lax.dot_general(x, x, dn, preferred_element_type=jnp.float32)
    t_ref[1] += lax.dot_general(x, xs1, dn, preferred_element_type=jnp.float32)
    t_ref[2] += lax.dot_general(x, xs2, dn, preferred_element_type=jnp.float32)
    t_ref[3] += lax.dot_general(xs1, xs2, dn, preferred_element_type=jnp.float32)


def _tc_moments(domain2d, feats):
    return pl.pallas_call(
        _mm_body,
        grid=(_B // _BLK,),
        in_specs=[
            pl.BlockSpec((_BLK, 1), lambda i: (i, 0)),
            pl.BlockSpec((_BLK, _D), lambda i: (i, 0)),
        ],
        out_specs=pl.BlockSpec((_M, _D, _D), lambda i: (0, 0, 0)),
        out_shape=jax.ShapeDtypeStruct((_M, _D, _D), jnp.float32),
        compiler_params=pltpu.CompilerParams(dimension_semantics=("arbitrary",)),
    )(domain2d, feats)


def _outer(v):
    return lax.dot_general(v, v, (((0,), (0,)), ((), ())),
                           preferred_element_type=jnp.float32)


def _ep_body(sums_ref, cnts_ref, t_ref, out_ref):
    sums = jnp.concatenate([sums_ref[0], sums_ref[1]],
                           axis=-1).reshape(_M, _CP, _D)
    cnts = (cnts_ref[...].reshape(_NW, _NSEGP // 128, 128).sum(axis=0)
            .reshape(_M, _CP // 128, 128).reshape(_M, _CP))

    csafe = jnp.maximum(cnts, 1.0)
    mu = sums / csafe[:, :, None]
    presf = (cnts > 0.0).astype(jnp.float32)

    anchors = (1.0 - _MOM) * mu * presf[:, :, None]
    ag = jnp.zeros((_CP, _D), jnp.float32)
    for d in range(_M):
        upd = _MOM * ag + (1.0 - _MOM) * mu[d]
        pd = presf[d][:, None]
        ag = pd * upd + (1.0 - pd) * ag
    per = jnp.sum((anchors - ag[None]) ** 2, axis=-1) / _D
    nvalid = jnp.sum(presf)
    caa = jnp.where(nvalid > 0,
                    jnp.sum(per * presf) / jnp.maximum(nvalid, 1.0),
                    0.0)

    t = t_ref[...]
    s2 = [(t[0] + sg1 * t[1] + sg2 * t[2] + sg1 * sg2 * t[3]) * 0.25
          for d in range(_M)
          for sg1, sg2 in [(2 * (d & 1) - 1, 2 * (d >> 1) - 1)]]
    s2_tot = t[0]

    tot = jnp.sum(sums, axis=(0, 1)).reshape(1, _D)
    mu_g = tot / _B
    cov = (s2_tot - _B * _outer(mu_g)) / (_B + 1e-6)
    rows = lax.broadcasted_iota(jnp.int32, (_D, _D), 0)
    cols = lax.broadcasted_iota(jnp.int32, (_D, _D), 1)
    eye = (rows == cols).astype(jnp.float32)
    g_mean = (1.0 - _MOM) * mu_g
    g_cov = _MOM * eye + (1.0 - _MOM) * cov

    loss = jnp.float32(0.0)
    nval = jnp.float32(0.0)
    for d in range(_M):
        cnt = jnp.sum(cnts[d])
        s_row = jnp.sum(sums[d], axis=0).reshape(1, _D)
        mu_d = s_row / jnp.maximum(cnt, 1.0)
        cov_d = (s2[d] - cnt * _outer(mu_d)) / (cnt + 1e-6)
        l_d = jnp.mean((mu_d - g_mean) ** 2) + jnp.mean((cov_d - g_cov) ** 2)
        has = (cnt > 0).astype(jnp.float32)
        loss = loss + has * l_d
        nval = nval + has
    stats = jnp.where(nval > 0, loss / jnp.maximum(nval, 1.0), 0.0)

    out_ref[...] = jnp.full((1, 1), caa + stats, jnp.float32)


def _tc_epilogue(sums_p, cnts_p, s2):
    return pl.pallas_call(
        _ep_body,
        out_shape=jax.ShapeDtypeStruct((1, 1), jnp.float32),
    )(sums_p, cnts_p, s2)


def kernel(feats, labels, domain_ids):
    s2 = _tc_moments(domain_ids.astype(jnp.int8).reshape(_B, 1), feats)
    feats_t = (feats.reshape(_B // 8, 8, _H, 128)
               .transpose(0, 2, 1, 3).reshape(_B * _H, 128))
    sums_p, cnts_p = _get_sc_segsum()(feats_t, labels, domain_ids)
    loss = _tc_epilogue(sums_p,
                        cnts_p.reshape(_NW * _NSEGP // 128, 128),
                        s2)
    return loss.reshape(())
